# Initial kernel scaffold; baseline (speedup 1.0000x reference)
import jax
import jax.numpy as jnp
from jax.experimental import pallas as pl


def kernel(feature, params, edge_index):
    raise NotImplementedError("write your pallas kernel here")


# speedup vs baseline: 11.3802x; 11.3802x over previous
import functools

import jax
import jax.numpy as jnp
from jax import lax
from jax.experimental import pallas as pl
from jax.experimental.pallas import tpu as pltpu
from jax.experimental.pallas import tpu_sc as plsc

N = 10000
E = 320000
_SC_PARAMS = pltpu.CompilerParams(needs_layout_passes=False)
_MESH = plsc.VectorSubcoreMesh(core_axis_name="c", subcore_axis_name="s")


def _gelu_exp(x, cvec):
    xa = jnp.abs(x) * 0.7071067811865476
    t = 1.0 / (1.0 + 0.3275911 * xa)
    poly = t * (0.254829592 + t * (-0.284496736 + t * (1.421413741
                + t * (-1.453152027 + t * 1.061405429))))
    erf = 1.0 - poly * jnp.exp(-xa * xa)
    erf = jnp.where(x >= 0, erf, -erf)
    g = 0.5 * x * (1.0 + erf)
    return jnp.exp(g - cvec)


def _make_sc_a(H):
    ET = E // 32
    CH = 2000
    NCH = ET // CH

    @functools.partial(
        pl.kernel,
        out_type=[jax.ShapeDtypeStruct((H * E,), jnp.float32),
                  jax.ShapeDtypeStruct((32 * H * N,), jnp.float32)],
        mesh=_MESH,
        compiler_params=_SC_PARAMS,
        scratch_types=[pltpu.VMEM((H * N,), jnp.float32),
                       pltpu.VMEM((H * N,), jnp.float32),
                       pltpu.VMEM((H * N,), jnp.float32),
                       pltpu.VMEM((CH,), jnp.int32),
                       pltpu.VMEM((CH,), jnp.int32),
                       pltpu.VMEM((H * CH,), jnp.float32),
                       pltpu.VMEM((H * 16,), jnp.float32)],
    )
    def sc_a(src_hbm, dst_hbm, s1_hbm, s2_hbm, c_hbm, ex_hbm, den_hbm,
             s1t, s2t, den, srcb, dstb, exb, cvb):
        cc = lax.axis_index("c")
        ss = lax.axis_index("s")
        wid = ss * 2 + cc
        pltpu.sync_copy(s1_hbm, s1t)
        pltpu.sync_copy(s2_hbm, s2t)
        pltpu.sync_copy(c_hbm, cvb)

        def zero_body(i, _):
            den[pl.ds(i * 16, 16)] = jnp.zeros((16,), jnp.float32)
            return 0
        lax.fori_loop(0, (H * N) // 16, zero_body, 0)

        base0 = wid * ET
        for ch in range(NCH):
            base = base0 + ch * CH
            pltpu.sync_copy(src_hbm.at[pl.ds(base, CH)], srcb)
            pltpu.sync_copy(dst_hbm.at[pl.ds(base, CH)], dstb)

            def e_body(i, _):
                off = i * 16
                sv = srcb[pl.ds(off, 16)]
                dv = dstb[pl.ds(off, 16)]
                for h in range(H):
                    a1 = plsc.load_gather(s1t, [sv + h * N])
                    a2 = plsc.load_gather(s2t, [dv + h * N])
                    ex = _gelu_exp(a1 + a2, cvb[pl.ds(h * 16, 16)])
                    exb[pl.ds(h * CH + off, 16)] = ex
                    plsc.addupdate_scatter(den, [dv + h * N], ex)
                return 0
            lax.fori_loop(0, CH // 16, e_body, 0)
            for h in range(H):
                pltpu.sync_copy(exb.at[pl.ds(h * CH, CH)],
                                ex_hbm.at[pl.ds(h * E + base, CH)])
        pltpu.sync_copy(den, den_hbm.at[pl.ds(wid * H * N, H * N)])

    return sc_a


def _make_sc_b(F, H):
    K = F // 4
    T = 32 // K
    ET = E // T
    CH = 2000
    NCH = ET // CH
    KH = K // H

    @functools.partial(
        pl.kernel,
        out_type=jax.ShapeDtypeStruct((T * F * N,), jnp.float32),
        mesh=_MESH,
        compiler_params=_SC_PARAMS,
        scratch_types=[pltpu.VMEM((4 * N,), jnp.float32),
                       pltpu.VMEM((4 * N,), jnp.float32),
                       pltpu.VMEM((CH,), jnp.int32),
                       pltpu.VMEM((CH,), jnp.int32),
                       pltpu.VMEM((CH,), jnp.float32)],
    )
    def sc_b(src_hbm, dst_hbm, ex_hbm, z_hbm, num_hbm,
             zc, acc, srcb, dstb, exb):
        cc = lax.axis_index("c")
        ss = lax.axis_index("s")
        wid = ss * 2 + cc
        chunk = wid // T
        part = wid % T
        f0 = chunk * 4
        head = chunk // KH
        pltpu.sync_copy(z_hbm.at[pl.ds(f0 * N, 4 * N)], zc)

        def zero_body(i, _):
            acc[pl.ds(i * 16, 16)] = jnp.zeros((16,), jnp.float32)
            return 0
        lax.fori_loop(0, (4 * N) // 16, zero_body, 0)

        ebase0 = part * ET

        def chunk_body(chi, _):
            base = ebase0 + chi * CH
            pltpu.sync_copy(src_hbm.at[pl.ds(base, CH)], srcb)
            pltpu.sync_copy(dst_hbm.at[pl.ds(base, CH)], dstb)
            pltpu.sync_copy(ex_hbm.at[pl.ds(head * E + base, CH)], exb)

            def e_body(i, _):
                off = i * 16
                sv = srcb[pl.ds(off, 16)]
                dv = dstb[pl.ds(off, 16)]
                w = exb[pl.ds(off, 16)]
                for r in range(4):
                    g = plsc.load_gather(zc, [sv + r * N])
                    plsc.addupdate_scatter(acc, [dv + r * N], g * w)
                return 0
            lax.fori_loop(0, CH // 16, e_body, 0)
            return 0
        lax.fori_loop(0, NCH, chunk_body, 0)
        pltpu.sync_copy(acc, num_hbm.at[pl.ds((part * F + f0) * N, 4 * N)])

    return sc_b


def _attn_scalars(z, a1, a2):
    s1 = lax.dot_general(a1, z, (((1,), (0,)), ((), ())),
                         preferred_element_type=jnp.float32)
    s2 = lax.dot_general(a2, z, (((1,), (0,)), ((), ())),
                         preferred_element_type=jnp.float32)
    c = jnp.clip(jnp.max(s1) + jnp.max(s2), 0.0, 30.0)
    return s1, s2, c


def _merge(num_parts, den_parts, F, H, T):
    num = jnp.sum(num_parts, axis=0)
    den = jnp.sum(den_parts, axis=0)
    den = jnp.maximum(den, 1e-16)
    FH = F // H
    dens = [jnp.broadcast_to(den[h:h + 1], (FH, N)) for h in range(H)]
    return num / jnp.concatenate(dens, axis=0)


def _dense_call(body, out_shapes, *inputs):
    return pl.pallas_call(
        body,
        out_shape=out_shapes,
    )(*inputs)


def kernel(feature, params, edge_index):
    src = edge_index[0]
    dst = edge_index[1]

    def head_wb(p):
        return p["W"], p["b"], p["a"][:, 0]

    def d0_body(feat_ref, w1_ref, b1_ref, a1_ref, w2_ref, b2_ref, a2_ref,
                z_ref, s1_ref, s2_ref, c_ref):
        feat = feat_ref[...]
        for h, (w_ref, b_ref, a_ref) in enumerate(
                ((w1_ref, b1_ref, a1_ref), (w2_ref, b2_ref, a2_ref))):
            z = lax.dot_general(w_ref[...], feat, (((0,), (1,)), ((), ())),
                                preferred_element_type=jnp.float32)
            z = z + b_ref[...][:, None]
            a = a_ref[...]
            s1, s2, c = _attn_scalars(z, a[:1], a[1:])
            z_ref[pl.ds(h * 64, 64), :] = z
            s1_ref[pl.ds(h, 1), :] = s1
            s2_ref[pl.ds(h, 1), :] = s2
            c_ref[pl.ds(h, 1), :] = jnp.full((1, 16), c, jnp.float32)

    l0h = params["l0"]["heads"]
    w1, b1, av1 = head_wb(l0h[0])
    w2, b2, av2 = head_wb(l0h[1])
    a1m = jnp.stack([av1[:64], av1[64:]])
    a2m = jnp.stack([av2[:64], av2[64:]])
    zT0, s1_0, s2_0, c0 = _dense_call(
        d0_body,
        [jax.ShapeDtypeStruct((128, N), jnp.float32),
         jax.ShapeDtypeStruct((2, N), jnp.float32),
         jax.ShapeDtypeStruct((2, N), jnp.float32),
         jax.ShapeDtypeStruct((2, 16), jnp.float32)],
        feature, w1, b1, a1m, w2, b2, a2m)

    sc_a2 = _make_sc_a(2)
    sc_a1 = _make_sc_a(1)
    sc_b128 = _make_sc_b(128, 2)
    sc_b64 = _make_sc_b(64, 1)
    sc_b16 = _make_sc_b(16, 2)

    ex0, denp0 = sc_a2(src, dst, s1_0.reshape(-1), s2_0.reshape(-1),
                       c0.reshape(-1))
    nump0 = sc_b128(src, dst, ex0, zT0.reshape(-1))

    def mid_body(F_in, H_in, T_in, FH_out):
        def body(nump_ref, denp_ref, w_ref, b_ref, a_ref,
                 z_ref, s1_ref, s2_ref, c_ref):
            h_in = _merge(nump_ref[...], denp_ref[...], F_in, H_in, T_in)
            z = lax.dot_general(w_ref[...], h_in, (((0,), (0,)), ((), ())),
                                preferred_element_type=jnp.float32)
            z = z + b_ref[...][:, None]
            a = a_ref[...]
            s1, s2, c = _attn_scalars(z, a[:1], a[1:])
            z_ref[...] = z
            s1_ref[...] = s1
            s2_ref[...] = s2
            c_ref[...] = jnp.full((1, 16), c, jnp.float32)
        return body

    def mid2_body(F_in, H_in, T_in, FH_out):
        def body(nump_ref, denp_ref, w1_ref, b1_ref, a1_ref,
                 w2_ref, b2_ref, a2_ref, z_ref, s1_ref, s2_ref, c_ref):
            h_in = _merge(nump_ref[...], denp_ref[...], F_in, H_in, T_in)
            for h, (w_ref, b_ref, a_ref) in enumerate(
                    ((w1_ref, b1_ref, a1_ref), (w2_ref, b2_ref, a2_ref))):
                z = lax.dot_general(w_ref[...], h_in,
                                    (((0,), (0,)), ((), ())),
                                    preferred_element_type=jnp.float32)
                z = z + b_ref[...][:, None]
                a = a_ref[...]
                s1, s2, c = _attn_scalars(z, a[:1], a[1:])
                z_ref[pl.ds(h * FH_out, FH_out), :] = z
                s1_ref[pl.ds(h, 1), :] = s1
                s2_ref[pl.ds(h, 1), :] = s2
                c_ref[pl.ds(h, 1), :] = jnp.full((1, 16), c, jnp.float32)
        return body

    l0o = params["l0"]["out"]
    wo, bo, avo = head_wb(l0o)
    aom = jnp.stack([avo[:64], avo[64:]])
    zT1, s1_1, s2_1, c1 = _dense_call(
        mid_body(128, 2, 1, 64),
        [jax.ShapeDtypeStruct((64, N), jnp.float32),
         jax.ShapeDtypeStruct((1, N), jnp.float32),
         jax.ShapeDtypeStruct((1, N), jnp.float32),
         jax.ShapeDtypeStruct((1, 16), jnp.float32)],
        nump0.reshape(1, 128, N), denp0.reshape(32, 2, N), wo, bo, aom)

    ex1, denp1 = sc_a1(src, dst, s1_1.reshape(-1), s2_1.reshape(-1),
                       c1.reshape(-1))
    nump1 = sc_b64(src, dst, ex1, zT1.reshape(-1))

    l1h = params["l1"]["heads"]
    w1, b1, av1 = head_wb(l1h[0])
    w2, b2, av2 = head_wb(l1h[1])
    a1m = jnp.stack([av1[:64], av1[64:]])
    a2m = jnp.stack([av2[:64], av2[64:]])
    zT2, s1_2, s2_2, c2 = _dense_call(
        mid2_body(64, 1, 2, 64),
        [jax.ShapeDtypeStruct((128, N), jnp.float32),
         jax.ShapeDtypeStruct((2, N), jnp.float32),
         jax.ShapeDtypeStruct((2, N), jnp.float32),
         jax.ShapeDtypeStruct((2, 16), jnp.float32)],
        nump1.reshape(2, 64, N), denp1.reshape(32, 1, N),
        w1, b1, a1m, w2, b2, a2m)

    ex2, denp2 = sc_a2(src, dst, s1_2.reshape(-1), s2_2.reshape(-1),
                       c2.reshape(-1))
    nump2 = sc_b128(src, dst, ex2, zT2.reshape(-1))

    l1o = params["l1"]["out"]
    wo, bo, avo = head_wb(l1o)
    aom = jnp.stack([avo[:64], avo[64:]])
    zT3, s1_3, s2_3, c3 = _dense_call(
        mid_body(128, 2, 1, 64),
        [jax.ShapeDtypeStruct((64, N), jnp.float32),
         jax.ShapeDtypeStruct((1, N), jnp.float32),
         jax.ShapeDtypeStruct((1, N), jnp.float32),
         jax.ShapeDtypeStruct((1, 16), jnp.float32)],
        nump2.reshape(1, 128, N), denp2.reshape(32, 2, N), wo, bo, aom)

    ex3, denp3 = sc_a1(src, dst, s1_3.reshape(-1), s2_3.reshape(-1),
                       c3.reshape(-1))
    nump3 = sc_b64(src, dst, ex3, zT3.reshape(-1))

    def d4_body(nump_ref, denp_ref, w1_ref, b1_ref, a1_ref,
                w2_ref, b2_ref, a2_ref, z_ref, s1_ref, s2_ref, c_ref):
        h_in = _merge(nump_ref[...], denp_ref[...], 64, 1, 2)
        for h, (w_ref, b_ref, a_ref) in enumerate(
                ((w1_ref, b1_ref, a1_ref), (w2_ref, b2_ref, a2_ref))):
            z = lax.dot_general(w_ref[...], h_in, (((0,), (0,)), ((), ())),
                                preferred_element_type=jnp.float32)
            z = z + b_ref[...][:, None]
            a = a_ref[...]
            s1, s2, c = _attn_scalars(z, a[:1], a[1:])
            z_ref[pl.ds(h * 8, 7), :] = z
            z_ref[pl.ds(h * 8 + 7, 1), :] = jnp.zeros((1, N), jnp.float32)
            s1_ref[pl.ds(h, 1), :] = s1
            s2_ref[pl.ds(h, 1), :] = s2
            c_ref[pl.ds(h, 1), :] = jnp.full((1, 16), c, jnp.float32)

    olh = params["outl"]["heads"]
    w1, b1, av1 = head_wb(olh[0])
    w2, b2, av2 = head_wb(olh[1])
    a1m = jnp.stack([av1[:7], av1[7:]])
    a2m = jnp.stack([av2[:7], av2[7:]])
    zT4, s1_4, s2_4, c4 = _dense_call(
        d4_body,
        [jax.ShapeDtypeStruct((16, N), jnp.float32),
         jax.ShapeDtypeStruct((2, N), jnp.float32),
         jax.ShapeDtypeStruct((2, N), jnp.float32),
         jax.ShapeDtypeStruct((2, 16), jnp.float32)],
        nump3.reshape(2, 64, N), denp3.reshape(32, 1, N),
        w1, b1, a1m, w2, b2, a2m)

    ex4, denp4 = sc_a2(src, dst, s1_4.reshape(-1), s2_4.reshape(-1),
                       c4.reshape(-1))
    nump4 = sc_b16(src, dst, ex4, zT4.reshape(-1))

    def d5_body(nump_ref, denp_ref, lw_ref, lb_ref, out_ref):
        num = jnp.sum(nump_ref[...], axis=0)
        den = jnp.sum(denp_ref[...], axis=0)
        den = jnp.maximum(den, 1e-16)
        o0 = num[0:7] / den[0:1]
        o1 = num[8:15] / den[1:2]
        hmean = 0.5 * (o0 + o1)
        logits = lax.dot_general(hmean, lw_ref[...], (((0,), (0,)), ((), ())),
                                 preferred_element_type=jnp.float32)
        logits = logits + lb_ref[...][None, :]
        m = jnp.max(logits, axis=1, keepdims=True)
        ez = jnp.exp(logits - m)
        out_ref[...] = ez / jnp.sum(ez, axis=1, keepdims=True)

    lin = params["outl"]["lin"]
    out = _dense_call(
        d5_body,
        jax.ShapeDtypeStruct((N, 7), jnp.float32),
        nump4.reshape(8, 16, N), denp4.reshape(32, 2, N),
        lin["W"], lin["b"])
    return out


# speedup vs baseline: 25.7632x; 2.2639x over previous
import functools

import jax
import jax.numpy as jnp
from jax import lax
from jax.experimental import pallas as pl
from jax.experimental.pallas import tpu as pltpu
from jax.experimental.pallas import tpu_sc as plsc

N = 10000
E = 320000
_SC_PARAMS = pltpu.CompilerParams(needs_layout_passes=False)
_MESH = plsc.VectorSubcoreMesh(core_axis_name="c", subcore_axis_name="s")


def _gelu_exp(x, cvec):
    xa = jnp.abs(x) * 0.7071067811865476
    t = 1.0 / (1.0 + 0.3275911 * xa)
    poly = t * (0.254829592 + t * (-0.284496736 + t * (1.421413741
                + t * (-1.453152027 + t * 1.061405429))))
    erf = 1.0 - poly * jnp.exp(-xa * xa)
    erf = jnp.where(x >= 0, erf, -erf)
    g = 0.5 * x * (1.0 + erf)
    return jnp.exp(g - cvec)


def _make_sc_a(H):
    ET = E // 32
    CH = 2000
    NCH = ET // CH

    @functools.partial(
        pl.kernel,
        out_type=[jax.ShapeDtypeStruct((H * E,), jnp.float32),
                  jax.ShapeDtypeStruct((32 * H * N,), jnp.float32)],
        mesh=_MESH,
        compiler_params=_SC_PARAMS,
        scratch_types=[pltpu.VMEM((H * N,), jnp.float32),
                       pltpu.VMEM((H * N,), jnp.float32),
                       pltpu.VMEM((H * N,), jnp.float32),
                       pltpu.VMEM((CH,), jnp.int32),
                       pltpu.VMEM((CH,), jnp.int32),
                       pltpu.VMEM((H * CH,), jnp.float32),
                       pltpu.VMEM((H * 16,), jnp.float32)],
    )
    def sc_a(src_hbm, dst_hbm, s1_hbm, s2_hbm, c_hbm, ex_hbm, den_hbm,
             s1t, s2t, den, srcb, dstb, exb, cvb):
        cc = lax.axis_index("c")
        ss = lax.axis_index("s")
        wid = ss * 2 + cc
        pltpu.sync_copy(s1_hbm, s1t)
        pltpu.sync_copy(s2_hbm, s2t)
        pltpu.sync_copy(c_hbm, cvb)

        @functools.partial(plsc.parallel_loop, 0, (H * N) // 16, unroll=8)
        def _(i):
            den[pl.ds(i * 16, 16)] = jnp.zeros((16,), jnp.float32)

        base0 = wid * ET
        for ch in range(NCH):
            base = base0 + ch * CH
            pltpu.sync_copy(src_hbm.at[pl.ds(base, CH)], srcb)
            pltpu.sync_copy(dst_hbm.at[pl.ds(base, CH)], dstb)

            def e_body(i, _):
                off = i * 16
                sv = srcb[pl.ds(off, 16)]
                dv = dstb[pl.ds(off, 16)]
                for h in range(H):
                    a1 = plsc.load_gather(s1t, [sv + h * N])
                    a2 = plsc.load_gather(s2t, [dv + h * N])
                    ex = _gelu_exp(a1 + a2, cvb[pl.ds(h * 16, 16)])
                    exb[pl.ds(h * CH + off, 16)] = ex
                    plsc.addupdate_scatter(den, [dv + h * N], ex)
                return 0
            lax.fori_loop(0, CH // 16, e_body, 0)
            for h in range(H):
                pltpu.sync_copy(exb.at[pl.ds(h * CH, CH)],
                                ex_hbm.at[pl.ds(h * E + base, CH)])
        pltpu.sync_copy(den, den_hbm.at[pl.ds(wid * H * N, H * N)])

    return sc_a


def _make_sc_b(F, H):
    K = F // 4
    T = 32 // K
    ET = E // T
    CH = 2000
    NCH = ET // CH
    KH = K // H

    @functools.partial(
        pl.kernel,
        out_type=jax.ShapeDtypeStruct((T * F * N,), jnp.float32),
        mesh=_MESH,
        compiler_params=_SC_PARAMS,
        scratch_types=[pltpu.VMEM((4 * N,), jnp.float32),
                       pltpu.VMEM((4 * N,), jnp.float32),
                       pltpu.VMEM((CH,), jnp.int32),
                       pltpu.VMEM((CH,), jnp.int32),
                       pltpu.VMEM((CH,), jnp.float32)],
    )
    def sc_b(src_hbm, dst_hbm, ex_hbm, z_hbm, num_hbm,
             zc, acc, srcb, dstb, exb):
        cc = lax.axis_index("c")
        ss = lax.axis_index("s")
        wid = ss * 2 + cc
        chunk = wid // T
        part = wid % T
        f0 = chunk * 4
        head = chunk // KH
        pltpu.sync_copy(z_hbm.at[pl.ds(f0 * N, 4 * N)], zc)

        @functools.partial(plsc.parallel_loop, 0, (4 * N) // 16, unroll=8)
        def _(i):
            acc[pl.ds(i * 16, 16)] = jnp.zeros((16,), jnp.float32)

        ebase0 = part * ET

        def chunk_body(chi, _):
            base = ebase0 + chi * CH
            pltpu.sync_copy(src_hbm.at[pl.ds(base, CH)], srcb)
            pltpu.sync_copy(dst_hbm.at[pl.ds(base, CH)], dstb)
            pltpu.sync_copy(ex_hbm.at[pl.ds(head * E + base, CH)], exb)

            @functools.partial(plsc.parallel_loop, 0, CH // 16, unroll=8)
            def _(i):
                off = i * 16
                sv = srcb[pl.ds(off, 16)]
                dv = dstb[pl.ds(off, 16)]
                w = exb[pl.ds(off, 16)]
                for r in range(4):
                    g = plsc.load_gather(zc, [sv + r * N])
                    plsc.addupdate_scatter(acc, [dv + r * N], g * w)
            return 0
        lax.fori_loop(0, NCH, chunk_body, 0)
        pltpu.sync_copy(acc, num_hbm.at[pl.ds((part * F + f0) * N, 4 * N)])

    return sc_b


def _attn_scalars(z, a1, a2):
    s1 = lax.dot_general(a1, z, (((1,), (0,)), ((), ())),
                         preferred_element_type=jnp.float32)
    s2 = lax.dot_general(a2, z, (((1,), (0,)), ((), ())),
                         preferred_element_type=jnp.float32)
    c = jnp.clip(jnp.max(s1) + jnp.max(s2), 0.0, 30.0)
    return s1, s2, c


def _merge(num_parts, den_parts, F, H, T):
    num = jnp.sum(num_parts, axis=0)
    den = jnp.sum(den_parts, axis=0)
    den = jnp.maximum(den, 1e-16)
    FH = F // H
    dens = [jnp.broadcast_to(den[h:h + 1], (FH, N)) for h in range(H)]
    return num / jnp.concatenate(dens, axis=0)


def _dense_call(body, out_shapes, *inputs):
    return pl.pallas_call(
        body,
        out_shape=out_shapes,
    )(*inputs)


def kernel(feature, params, edge_index):
    src = edge_index[0]
    dst = edge_index[1]

    def head_wb(p):
        return p["W"], p["b"], p["a"][:, 0]

    def d0_body(feat_ref, w1_ref, b1_ref, a1_ref, w2_ref, b2_ref, a2_ref,
                z_ref, s1_ref, s2_ref, c_ref):
        feat = feat_ref[...]
        for h, (w_ref, b_ref, a_ref) in enumerate(
                ((w1_ref, b1_ref, a1_ref), (w2_ref, b2_ref, a2_ref))):
            z = lax.dot_general(w_ref[...], feat, (((0,), (1,)), ((), ())),
                                preferred_element_type=jnp.float32)
            z = z + b_ref[...][:, None]
            a = a_ref[...]
            s1, s2, c = _attn_scalars(z, a[:1], a[1:])
            z_ref[pl.ds(h * 64, 64), :] = z
            s1_ref[pl.ds(h, 1), :] = s1
            s2_ref[pl.ds(h, 1), :] = s2
            c_ref[pl.ds(h, 1), :] = jnp.full((1, 16), c, jnp.float32)

    l0h = params["l0"]["heads"]
    w1, b1, av1 = head_wb(l0h[0])
    w2, b2, av2 = head_wb(l0h[1])
    a1m = jnp.stack([av1[:64], av1[64:]])
    a2m = jnp.stack([av2[:64], av2[64:]])
    zT0, s1_0, s2_0, c0 = _dense_call(
        d0_body,
        [jax.ShapeDtypeStruct((128, N), jnp.float32),
         jax.ShapeDtypeStruct((2, N), jnp.float32),
         jax.ShapeDtypeStruct((2, N), jnp.float32),
         jax.ShapeDtypeStruct((2, 16), jnp.float32)],
        feature, w1, b1, a1m, w2, b2, a2m)

    sc_a2 = _make_sc_a(2)
    sc_a1 = _make_sc_a(1)
    sc_b128 = _make_sc_b(128, 2)
    sc_b64 = _make_sc_b(64, 1)
    sc_b16 = _make_sc_b(16, 2)

    ex0, denp0 = sc_a2(src, dst, s1_0.reshape(-1), s2_0.reshape(-1),
                       c0.reshape(-1))
    nump0 = sc_b128(src, dst, ex0, zT0.reshape(-1))

    def mid_body(F_in, H_in, T_in, FH_out):
        def body(nump_ref, denp_ref, w_ref, b_ref, a_ref,
                 z_ref, s1_ref, s2_ref, c_ref):
            h_in = _merge(nump_ref[...], denp_ref[...], F_in, H_in, T_in)
            z = lax.dot_general(w_ref[...], h_in, (((0,), (0,)), ((), ())),
                                preferred_element_type=jnp.float32)
            z = z + b_ref[...][:, None]
            a = a_ref[...]
            s1, s2, c = _attn_scalars(z, a[:1], a[1:])
            z_ref[...] = z
            s1_ref[...] = s1
            s2_ref[...] = s2
            c_ref[...] = jnp.full((1, 16), c, jnp.float32)
        return body

    def mid2_body(F_in, H_in, T_in, FH_out):
        def body(nump_ref, denp_ref, w1_ref, b1_ref, a1_ref,
                 w2_ref, b2_ref, a2_ref, z_ref, s1_ref, s2_ref, c_ref):
            h_in = _merge(nump_ref[...], denp_ref[...], F_in, H_in, T_in)
            for h, (w_ref, b_ref, a_ref) in enumerate(
                    ((w1_ref, b1_ref, a1_ref), (w2_ref, b2_ref, a2_ref))):
                z = lax.dot_general(w_ref[...], h_in,
                                    (((0,), (0,)), ((), ())),
                                    preferred_element_type=jnp.float32)
                z = z + b_ref[...][:, None]
                a = a_ref[...]
                s1, s2, c = _attn_scalars(z, a[:1], a[1:])
                z_ref[pl.ds(h * FH_out, FH_out), :] = z
                s1_ref[pl.ds(h, 1), :] = s1
                s2_ref[pl.ds(h, 1), :] = s2
                c_ref[pl.ds(h, 1), :] = jnp.full((1, 16), c, jnp.float32)
        return body

    l0o = params["l0"]["out"]
    wo, bo, avo = head_wb(l0o)
    aom = jnp.stack([avo[:64], avo[64:]])
    zT1, s1_1, s2_1, c1 = _dense_call(
        mid_body(128, 2, 1, 64),
        [jax.ShapeDtypeStruct((64, N), jnp.float32),
         jax.ShapeDtypeStruct((1, N), jnp.float32),
         jax.ShapeDtypeStruct((1, N), jnp.float32),
         jax.ShapeDtypeStruct((1, 16), jnp.float32)],
        nump0.reshape(1, 128, N), denp0.reshape(32, 2, N), wo, bo, aom)

    ex1, denp1 = sc_a1(src, dst, s1_1.reshape(-1), s2_1.reshape(-1),
                       c1.reshape(-1))
    nump1 = sc_b64(src, dst, ex1, zT1.reshape(-1))

    l1h = params["l1"]["heads"]
    w1, b1, av1 = head_wb(l1h[0])
    w2, b2, av2 = head_wb(l1h[1])
    a1m = jnp.stack([av1[:64], av1[64:]])
    a2m = jnp.stack([av2[:64], av2[64:]])
    zT2, s1_2, s2_2, c2 = _dense_call(
        mid2_body(64, 1, 2, 64),
        [jax.ShapeDtypeStruct((128, N), jnp.float32),
         jax.ShapeDtypeStruct((2, N), jnp.float32),
         jax.ShapeDtypeStruct((2, N), jnp.float32),
         jax.ShapeDtypeStruct((2, 16), jnp.float32)],
        nump1.reshape(2, 64, N), denp1.reshape(32, 1, N),
        w1, b1, a1m, w2, b2, a2m)

    ex2, denp2 = sc_a2(src, dst, s1_2.reshape(-1), s2_2.reshape(-1),
                       c2.reshape(-1))
    nump2 = sc_b128(src, dst, ex2, zT2.reshape(-1))

    l1o = params["l1"]["out"]
    wo, bo, avo = head_wb(l1o)
    aom = jnp.stack([avo[:64], avo[64:]])
    zT3, s1_3, s2_3, c3 = _dense_call(
        mid_body(128, 2, 1, 64),
        [jax.ShapeDtypeStruct((64, N), jnp.float32),
         jax.ShapeDtypeStruct((1, N), jnp.float32),
         jax.ShapeDtypeStruct((1, N), jnp.float32),
         jax.ShapeDtypeStruct((1, 16), jnp.float32)],
        nump2.reshape(1, 128, N), denp2.reshape(32, 2, N), wo, bo, aom)

    ex3, denp3 = sc_a1(src, dst, s1_3.reshape(-1), s2_3.reshape(-1),
                       c3.reshape(-1))
    nump3 = sc_b64(src, dst, ex3, zT3.reshape(-1))

    def d4_body(nump_ref, denp_ref, w1_ref, b1_ref, a1_ref,
                w2_ref, b2_ref, a2_ref, z_ref, s1_ref, s2_ref, c_ref):
        h_in = _merge(nump_ref[...], denp_ref[...], 64, 1, 2)
        for h, (w_ref, b_ref, a_ref) in enumerate(
                ((w1_ref, b1_ref, a1_ref), (w2_ref, b2_ref, a2_ref))):
            z = lax.dot_general(w_ref[...], h_in, (((0,), (0,)), ((), ())),
                                preferred_element_type=jnp.float32)
            z = z + b_ref[...][:, None]
            a = a_ref[...]
            s1, s2, c = _attn_scalars(z, a[:1], a[1:])
            z_ref[pl.ds(h * 8, 7), :] = z
            z_ref[pl.ds(h * 8 + 7, 1), :] = jnp.zeros((1, N), jnp.float32)
            s1_ref[pl.ds(h, 1), :] = s1
            s2_ref[pl.ds(h, 1), :] = s2
            c_ref[pl.ds(h, 1), :] = jnp.full((1, 16), c, jnp.float32)

    olh = params["outl"]["heads"]
    w1, b1, av1 = head_wb(olh[0])
    w2, b2, av2 = head_wb(olh[1])
    a1m = jnp.stack([av1[:7], av1[7:]])
    a2m = jnp.stack([av2[:7], av2[7:]])
    zT4, s1_4, s2_4, c4 = _dense_call(
        d4_body,
        [jax.ShapeDtypeStruct((16, N), jnp.float32),
         jax.ShapeDtypeStruct((2, N), jnp.float32),
         jax.ShapeDtypeStruct((2, N), jnp.float32),
         jax.ShapeDtypeStruct((2, 16), jnp.float32)],
        nump3.reshape(2, 64, N), denp3.reshape(32, 1, N),
        w1, b1, a1m, w2, b2, a2m)

    ex4, denp4 = sc_a2(src, dst, s1_4.reshape(-1), s2_4.reshape(-1),
                       c4.reshape(-1))
    nump4 = sc_b16(src, dst, ex4, zT4.reshape(-1))

    def d5_body(nump_ref, denp_ref, lw_ref, lb_ref, out_ref):
        num = jnp.sum(nump_ref[...], axis=0)
        den = jnp.sum(denp_ref[...], axis=0)
        den = jnp.maximum(den, 1e-16)
        o0 = num[0:7] / den[0:1]
        o1 = num[8:15] / den[1:2]
        hmean = 0.5 * (o0 + o1)
        logits = lax.dot_general(hmean, lw_ref[...], (((0,), (0,)), ((), ())),
                                 preferred_element_type=jnp.float32)
        logits = logits + lb_ref[...][None, :]
        m = jnp.max(logits, axis=1, keepdims=True)
        ez = jnp.exp(logits - m)
        out_ref[...] = ez / jnp.sum(ez, axis=1, keepdims=True)

    lin = params["outl"]["lin"]
    out = _dense_call(
        d5_body,
        jax.ShapeDtypeStruct((N, 7), jnp.float32),
        nump4.reshape(8, 16, N), denp4.reshape(32, 2, N),
        lin["W"], lin["b"])
    return out


# speedup vs baseline: 52.8903x; 2.0529x over previous
import functools

import jax
import jax.numpy as jnp
from jax import lax
from jax.experimental import pallas as pl
from jax.experimental.pallas import tpu as pltpu
from jax.experimental.pallas import tpu_sc as plsc

N = 10000
E = 320000
_SC_PARAMS = pltpu.CompilerParams(needs_layout_passes=False)
_MESH = plsc.VectorSubcoreMesh(core_axis_name="c", subcore_axis_name="s")


def _gelu_exp(x, cvec):
    xa = jnp.abs(x) * 0.7071067811865476
    t = 1.0 / (1.0 + 0.3275911 * xa)
    poly = t * (0.254829592 + t * (-0.284496736 + t * (1.421413741
                + t * (-1.453152027 + t * 1.061405429))))
    erf = 1.0 - poly * jnp.exp(-xa * xa)
    erf = jnp.where(x >= 0, erf, -erf)
    g = 0.5 * x * (1.0 + erf)
    return jnp.exp(g - cvec)


def _make_sc_a(H):
    ET = E // 32
    CH = 2000
    NCH = ET // CH

    @functools.partial(
        pl.kernel,
        out_type=[jax.ShapeDtypeStruct((H * E,), jnp.float32),
                  jax.ShapeDtypeStruct((32 * H * N,), jnp.float32)],
        mesh=_MESH,
        compiler_params=_SC_PARAMS,
        scratch_types=[pltpu.VMEM((H * N,), jnp.float32),
                       pltpu.VMEM((H * N,), jnp.float32),
                       pltpu.VMEM((H * N,), jnp.float32),
                       pltpu.VMEM((CH,), jnp.int32),
                       pltpu.VMEM((CH,), jnp.int32),
                       pltpu.VMEM((H * CH,), jnp.float32),
                       pltpu.VMEM((H * 16,), jnp.float32)],
    )
    def sc_a(src_hbm, dst_hbm, s1_hbm, s2_hbm, c_hbm, ex_hbm, den_hbm,
             s1t, s2t, den, srcb, dstb, exb, cvb):
        cc = lax.axis_index("c")
        ss = lax.axis_index("s")
        wid = ss * 2 + cc
        pltpu.sync_copy(s1_hbm, s1t)
        pltpu.sync_copy(s2_hbm, s2t)
        pltpu.sync_copy(c_hbm, cvb)

        @functools.partial(plsc.parallel_loop, 0, (H * N) // 16, unroll=8)
        def _(i):
            den[pl.ds(i * 16, 16)] = jnp.zeros((16,), jnp.float32)

        base0 = wid * ET
        for ch in range(NCH):
            base = base0 + ch * CH
            pltpu.sync_copy(src_hbm.at[pl.ds(base, CH)], srcb)
            pltpu.sync_copy(dst_hbm.at[pl.ds(base, CH)], dstb)

            def e_body(i, _):
                off = i * 16
                sv = srcb[pl.ds(off, 16)]
                dv = dstb[pl.ds(off, 16)]
                for h in range(H):
                    a1 = plsc.load_gather(s1t, [sv + h * N])
                    a2 = plsc.load_gather(s2t, [dv + h * N])
                    ex = _gelu_exp(a1 + a2, cvb[pl.ds(h * 16, 16)])
                    exb[pl.ds(h * CH + off, 16)] = ex
                    plsc.addupdate_scatter(den, [dv + h * N], ex)
                return 0
            lax.fori_loop(0, CH // 16, e_body, 0)
            for h in range(H):
                pltpu.sync_copy(exb.at[pl.ds(h * CH, CH)],
                                ex_hbm.at[pl.ds(h * E + base, CH)])
        pltpu.sync_copy(den, den_hbm.at[pl.ds(wid * H * N, H * N)])

    return sc_a


def _make_sc_b(F, H):
    K = F // 4
    T = 32 // K
    ET = E // T
    CH = 4000
    NCH = ET // CH
    KH = K // H
    assert NCH % 2 == 0

    @functools.partial(
        pl.kernel,
        out_type=jax.ShapeDtypeStruct((T * F * N,), jnp.float32),
        mesh=_MESH,
        compiler_params=_SC_PARAMS,
        scratch_types=[pltpu.VMEM((4 * N,), jnp.float32),
                       pltpu.VMEM((4 * N,), jnp.float32),
                       pltpu.VMEM((CH,), jnp.int32),
                       pltpu.VMEM((CH,), jnp.int32),
                       pltpu.VMEM((CH,), jnp.int32),
                       pltpu.VMEM((CH,), jnp.int32),
                       pltpu.VMEM((CH,), jnp.float32),
                       pltpu.VMEM((CH,), jnp.float32),
                       pltpu.SemaphoreType.DMA,
                       pltpu.SemaphoreType.DMA],
    )
    def sc_b(src_hbm, dst_hbm, ex_hbm, z_hbm, num_hbm,
             zc, acc, srcb0, srcb1, dstb0, dstb1, exb0, exb1, sem0, sem1):
        cc = lax.axis_index("c")
        ss = lax.axis_index("s")
        wid = ss * 2 + cc
        chunk = wid // T
        part = wid % T
        f0 = chunk * 4
        head = chunk // KH
        srcbufs = (srcb0, srcb1)
        dstbufs = (dstb0, dstb1)
        exbufs = (exb0, exb1)
        sems = (sem0, sem1)
        ebase0 = part * ET

        def start(cidx, p):
            base = ebase0 + cidx * CH
            pltpu.async_copy(src_hbm.at[pl.ds(base, CH)], srcbufs[p], sems[p])
            pltpu.async_copy(dst_hbm.at[pl.ds(base, CH)], dstbufs[p], sems[p])
            pltpu.async_copy(ex_hbm.at[pl.ds(head * E + base, CH)],
                             exbufs[p], sems[p])

        def drain(cidx, p):
            base = ebase0 + cidx * CH
            pltpu.make_async_copy(src_hbm.at[pl.ds(base, CH)],
                                  srcbufs[p], sems[p]).wait()
            pltpu.make_async_copy(dst_hbm.at[pl.ds(base, CH)],
                                  dstbufs[p], sems[p]).wait()
            pltpu.make_async_copy(ex_hbm.at[pl.ds(head * E + base, CH)],
                                  exbufs[p], sems[p]).wait()

        start(0, 0)
        start(1, 1)
        pltpu.sync_copy(z_hbm.at[pl.ds(f0 * N, 4 * N)], zc)

        @functools.partial(plsc.parallel_loop, 0, (4 * N) // 16, unroll=8)
        def _(i):
            acc[pl.ds(i * 16, 16)] = jnp.zeros((16,), jnp.float32)

        def compute(p):
            srcb, dstb, exb = srcbufs[p], dstbufs[p], exbufs[p]

            @functools.partial(plsc.parallel_loop, 0, CH // 16, unroll=8)
            def _(i):
                off = i * 16
                sv = srcb[pl.ds(off, 16)]
                dv = dstb[pl.ds(off, 16)]
                w = exb[pl.ds(off, 16)]
                for r in range(4):
                    gth = plsc.load_gather(zc, [sv + r * N])
                    plsc.addupdate_scatter(acc, [dv + r * N], gth * w)

        def chunk_body(g, _):
            for p in range(2):
                cidx = g * 2 + p
                drain(cidx, p)
                compute(p)
                start(cidx + 2, p)
            return 0
        lax.fori_loop(0, NCH // 2 - 1, chunk_body, 0)
        for p in range(2):
            drain(NCH - 2 + p, p)
            compute(p)
        pltpu.sync_copy(acc, num_hbm.at[pl.ds((part * F + f0) * N, 4 * N)])

    return sc_b


def _attn_scalars(z, a1, a2):
    s1 = lax.dot_general(a1, z, (((1,), (0,)), ((), ())),
                         preferred_element_type=jnp.float32)
    s2 = lax.dot_general(a2, z, (((1,), (0,)), ((), ())),
                         preferred_element_type=jnp.float32)
    c = jnp.clip(jnp.max(s1) + jnp.max(s2), 0.0, 30.0)
    return s1, s2, c


def _merge(num_parts, den_parts, F, H, T):
    num = jnp.sum(num_parts, axis=0)
    den = jnp.sum(den_parts, axis=0)
    den = jnp.maximum(den, 1e-16)
    FH = F // H
    dens = [jnp.broadcast_to(den[h:h + 1], (FH, N)) for h in range(H)]
    return num / jnp.concatenate(dens, axis=0)


def _dense_call(body, out_shapes, *inputs):
    return pl.pallas_call(
        body,
        out_shape=out_shapes,
    )(*inputs)


def kernel(feature, params, edge_index):
    src = edge_index[0]
    dst = edge_index[1]

    def head_wb(p):
        return p["W"], p["b"], p["a"][:, 0]

    def d0_body(feat_ref, w1_ref, b1_ref, a1_ref, w2_ref, b2_ref, a2_ref,
                z_ref, s1_ref, s2_ref, c_ref):
        feat = feat_ref[...]
        for h, (w_ref, b_ref, a_ref) in enumerate(
                ((w1_ref, b1_ref, a1_ref), (w2_ref, b2_ref, a2_ref))):
            z = lax.dot_general(w_ref[...], feat, (((0,), (1,)), ((), ())),
                                preferred_element_type=jnp.float32)
            z = z + b_ref[...][:, None]
            a = a_ref[...]
            s1, s2, c = _attn_scalars(z, a[:1], a[1:])
            z_ref[pl.ds(h * 64, 64), :] = z
            s1_ref[pl.ds(h, 1), :] = s1
            s2_ref[pl.ds(h, 1), :] = s2
            c_ref[pl.ds(h, 1), :] = jnp.full((1, 16), c, jnp.float32)

    l0h = params["l0"]["heads"]
    w1, b1, av1 = head_wb(l0h[0])
    w2, b2, av2 = head_wb(l0h[1])
    a1m = jnp.stack([av1[:64], av1[64:]])
    a2m = jnp.stack([av2[:64], av2[64:]])
    zT0, s1_0, s2_0, c0 = _dense_call(
        d0_body,
        [jax.ShapeDtypeStruct((128, N), jnp.float32),
         jax.ShapeDtypeStruct((2, N), jnp.float32),
         jax.ShapeDtypeStruct((2, N), jnp.float32),
         jax.ShapeDtypeStruct((2, 16), jnp.float32)],
        feature, w1, b1, a1m, w2, b2, a2m)

    sc_a2 = _make_sc_a(2)
    sc_a1 = _make_sc_a(1)
    sc_b128 = _make_sc_b(128, 2)
    sc_b64 = _make_sc_b(64, 1)
    sc_b16 = _make_sc_b(16, 2)

    ex0, denp0 = sc_a2(src, dst, s1_0.reshape(-1), s2_0.reshape(-1),
                       c0.reshape(-1))
    nump0 = sc_b128(src, dst, ex0, zT0.reshape(-1))

    def mid_body(F_in, H_in, T_in, FH_out):
        def body(nump_ref, denp_ref, w_ref, b_ref, a_ref,
                 z_ref, s1_ref, s2_ref, c_ref):
            h_in = _merge(nump_ref[...], denp_ref[...], F_in, H_in, T_in)
            z = lax.dot_general(w_ref[...], h_in, (((0,), (0,)), ((), ())),
                                preferred_element_type=jnp.float32)
            z = z + b_ref[...][:, None]
            a = a_ref[...]
            s1, s2, c = _attn_scalars(z, a[:1], a[1:])
            z_ref[...] = z
            s1_ref[...] = s1
            s2_ref[...] = s2
            c_ref[...] = jnp.full((1, 16), c, jnp.float32)
        return body

    def mid2_body(F_in, H_in, T_in, FH_out):
        def body(nump_ref, denp_ref, w1_ref, b1_ref, a1_ref,
                 w2_ref, b2_ref, a2_ref, z_ref, s1_ref, s2_ref, c_ref):
            h_in = _merge(nump_ref[...], denp_ref[...], F_in, H_in, T_in)
            for h, (w_ref, b_ref, a_ref) in enumerate(
                    ((w1_ref, b1_ref, a1_ref), (w2_ref, b2_ref, a2_ref))):
                z = lax.dot_general(w_ref[...], h_in,
                                    (((0,), (0,)), ((), ())),
                                    preferred_element_type=jnp.float32)
                z = z + b_ref[...][:, None]
                a = a_ref[...]
                s1, s2, c = _attn_scalars(z, a[:1], a[1:])
                z_ref[pl.ds(h * FH_out, FH_out), :] = z
                s1_ref[pl.ds(h, 1), :] = s1
                s2_ref[pl.ds(h, 1), :] = s2
                c_ref[pl.ds(h, 1), :] = jnp.full((1, 16), c, jnp.float32)
        return body

    l0o = params["l0"]["out"]
    wo, bo, avo = head_wb(l0o)
    aom = jnp.stack([avo[:64], avo[64:]])
    zT1, s1_1, s2_1, c1 = _dense_call(
        mid_body(128, 2, 1, 64),
        [jax.ShapeDtypeStruct((64, N), jnp.float32),
         jax.ShapeDtypeStruct((1, N), jnp.float32),
         jax.ShapeDtypeStruct((1, N), jnp.float32),
         jax.ShapeDtypeStruct((1, 16), jnp.float32)],
        nump0.reshape(1, 128, N), denp0.reshape(32, 2, N), wo, bo, aom)

    ex1, denp1 = sc_a1(src, dst, s1_1.reshape(-1), s2_1.reshape(-1),
                       c1.reshape(-1))
    nump1 = sc_b64(src, dst, ex1, zT1.reshape(-1))

    l1h = params["l1"]["heads"]
    w1, b1, av1 = head_wb(l1h[0])
    w2, b2, av2 = head_wb(l1h[1])
    a1m = jnp.stack([av1[:64], av1[64:]])
    a2m = jnp.stack([av2[:64], av2[64:]])
    zT2, s1_2, s2_2, c2 = _dense_call(
        mid2_body(64, 1, 2, 64),
        [jax.ShapeDtypeStruct((128, N), jnp.float32),
         jax.ShapeDtypeStruct((2, N), jnp.float32),
         jax.ShapeDtypeStruct((2, N), jnp.float32),
         jax.ShapeDtypeStruct((2, 16), jnp.float32)],
        nump1.reshape(2, 64, N), denp1.reshape(32, 1, N),
        w1, b1, a1m, w2, b2, a2m)

    ex2, denp2 = sc_a2(src, dst, s1_2.reshape(-1), s2_2.reshape(-1),
                       c2.reshape(-1))
    nump2 = sc_b128(src, dst, ex2, zT2.reshape(-1))

    l1o = params["l1"]["out"]
    wo, bo, avo = head_wb(l1o)
    aom = jnp.stack([avo[:64], avo[64:]])
    zT3, s1_3, s2_3, c3 = _dense_call(
        mid_body(128, 2, 1, 64),
        [jax.ShapeDtypeStruct((64, N), jnp.float32),
         jax.ShapeDtypeStruct((1, N), jnp.float32),
         jax.ShapeDtypeStruct((1, N), jnp.float32),
         jax.ShapeDtypeStruct((1, 16), jnp.float32)],
        nump2.reshape(1, 128, N), denp2.reshape(32, 2, N), wo, bo, aom)

    ex3, denp3 = sc_a1(src, dst, s1_3.reshape(-1), s2_3.reshape(-1),
                       c3.reshape(-1))
    nump3 = sc_b64(src, dst, ex3, zT3.reshape(-1))

    def d4_body(nump_ref, denp_ref, w1_ref, b1_ref, a1_ref,
                w2_ref, b2_ref, a2_ref, z_ref, s1_ref, s2_ref, c_ref):
        h_in = _merge(nump_ref[...], denp_ref[...], 64, 1, 2)
        for h, (w_ref, b_ref, a_ref) in enumerate(
                ((w1_ref, b1_ref, a1_ref), (w2_ref, b2_ref, a2_ref))):
            z = lax.dot_general(w_ref[...], h_in, (((0,), (0,)), ((), ())),
                                preferred_element_type=jnp.float32)
            z = z + b_ref[...][:, None]
            a = a_ref[...]
            s1, s2, c = _attn_scalars(z, a[:1], a[1:])
            z_ref[pl.ds(h * 8, 7), :] = z
            z_ref[pl.ds(h * 8 + 7, 1), :] = jnp.zeros((1, N), jnp.float32)
            s1_ref[pl.ds(h, 1), :] = s1
            s2_ref[pl.ds(h, 1), :] = s2
            c_ref[pl.ds(h, 1), :] = jnp.full((1, 16), c, jnp.float32)

    olh = params["outl"]["heads"]
    w1, b1, av1 = head_wb(olh[0])
    w2, b2, av2 = head_wb(olh[1])
    a1m = jnp.stack([av1[:7], av1[7:]])
    a2m = jnp.stack([av2[:7], av2[7:]])
    zT4, s1_4, s2_4, c4 = _dense_call(
        d4_body,
        [jax.ShapeDtypeStruct((16, N), jnp.float32),
         jax.ShapeDtypeStruct((2, N), jnp.float32),
         jax.ShapeDtypeStruct((2, N), jnp.float32),
         jax.ShapeDtypeStruct((2, 16), jnp.float32)],
        nump3.reshape(2, 64, N), denp3.reshape(32, 1, N),
        w1, b1, a1m, w2, b2, a2m)

    ex4, denp4 = sc_a2(src, dst, s1_4.reshape(-1), s2_4.reshape(-1),
                       c4.reshape(-1))
    nump4 = sc_b16(src, dst, ex4, zT4.reshape(-1))

    def d5_body(nump_ref, denp_ref, lw_ref, lb_ref, out_ref):
        num = jnp.sum(nump_ref[...], axis=0)
        den = jnp.sum(denp_ref[...], axis=0)
        den = jnp.maximum(den, 1e-16)
        o0 = num[0:7] / den[0:1]
        o1 = num[8:15] / den[1:2]
        hmean = 0.5 * (o0 + o1)
        logits = lax.dot_general(hmean, lw_ref[...], (((0,), (0,)), ((), ())),
                                 preferred_element_type=jnp.float32)
        logits = logits + lb_ref[...][None, :]
        m = jnp.max(logits, axis=1, keepdims=True)
        ez = jnp.exp(logits - m)
        out_ref[...] = ez / jnp.sum(ez, axis=1, keepdims=True)

    lin = params["outl"]["lin"]
    out = _dense_call(
        d5_body,
        jax.ShapeDtypeStruct((N, 7), jnp.float32),
        nump4.reshape(8, 16, N), denp4.reshape(32, 2, N),
        lin["W"], lin["b"])
    return out


# speedup vs baseline: 78.3246x; 1.4809x over previous
import functools

import jax
import jax.numpy as jnp
from jax import lax
from jax.experimental import pallas as pl
from jax.experimental.pallas import tpu as pltpu
from jax.experimental.pallas import tpu_sc as plsc

N = 10000
E = 320000
_SC_PARAMS = pltpu.CompilerParams(needs_layout_passes=False)
_MESH = plsc.VectorSubcoreMesh(core_axis_name="c", subcore_axis_name="s")


def _gelu_exp(x, cvec):
    xa = jnp.abs(x) * 0.7071067811865476
    t = 1.0 / (1.0 + 0.3275911 * xa)
    poly = t * (0.254829592 + t * (-0.284496736 + t * (1.421413741
                + t * (-1.453152027 + t * 1.061405429))))
    erf = 1.0 - poly * jnp.exp(-xa * xa)
    erf = jnp.where(x >= 0, erf, -erf)
    g = 0.5 * x * (1.0 + erf)
    return jnp.exp(g - cvec)


def _make_sc_a(H):
    ET = E // 32
    CH = 2000
    NCH = ET // CH

    @functools.partial(
        pl.kernel,
        out_type=[jax.ShapeDtypeStruct((H * E,), jnp.float32),
                  jax.ShapeDtypeStruct((32 * H * N,), jnp.float32)],
        mesh=_MESH,
        compiler_params=_SC_PARAMS,
        scratch_types=[pltpu.VMEM((H * N,), jnp.float32),
                       pltpu.VMEM((H * N,), jnp.float32),
                       pltpu.VMEM((H * N,), jnp.float32),
                       pltpu.VMEM((CH,), jnp.int32),
                       pltpu.VMEM((CH,), jnp.int32),
                       pltpu.VMEM((CH,), jnp.int32),
                       pltpu.VMEM((CH,), jnp.int32),
                       pltpu.VMEM((H * CH,), jnp.float32),
                       pltpu.VMEM((H * CH,), jnp.float32),
                       pltpu.VMEM((H * 16,), jnp.float32),
                       pltpu.SemaphoreType.DMA,
                       pltpu.SemaphoreType.DMA,
                       pltpu.SemaphoreType.DMA,
                       pltpu.SemaphoreType.DMA],
    )
    def sc_a(src_hbm, dst_hbm, s1_hbm, s2_hbm, c_hbm, ex_hbm, den_hbm,
             s1t, s2t, den, srcb0, srcb1, dstb0, dstb1, exb0, exb1, cvb,
             semi0, semi1, semo0, semo1):
        cc = lax.axis_index("c")
        ss = lax.axis_index("s")
        wid = ss * 2 + cc
        srcbufs = (srcb0, srcb1)
        dstbufs = (dstb0, dstb1)
        exbufs = (exb0, exb1)
        semis = (semi0, semi1)
        semos = (semo0, semo1)
        base0 = wid * ET

        def start_in(cidx, p):
            base = base0 + cidx * CH
            pltpu.async_copy(src_hbm.at[pl.ds(base, CH)], srcbufs[p],
                             semis[p])
            pltpu.async_copy(dst_hbm.at[pl.ds(base, CH)], dstbufs[p],
                             semis[p])

        def drain_in(cidx, p):
            base = base0 + cidx * CH
            pltpu.make_async_copy(src_hbm.at[pl.ds(base, CH)], srcbufs[p],
                                  semis[p]).wait()
            pltpu.make_async_copy(dst_hbm.at[pl.ds(base, CH)], dstbufs[p],
                                  semis[p]).wait()

        def start_out(cidx, p):
            base = base0 + cidx * CH
            for h in range(H):
                pltpu.async_copy(exbufs[p].at[pl.ds(h * CH, CH)],
                                 ex_hbm.at[pl.ds(h * E + base, CH)],
                                 semos[p])

        def drain_out(cidx, p):
            base = base0 + cidx * CH
            for h in range(H):
                pltpu.make_async_copy(exbufs[p].at[pl.ds(h * CH, CH)],
                                      ex_hbm.at[pl.ds(h * E + base, CH)],
                                      semos[p]).wait()

        start_in(0, 0)
        start_in(1, 1)
        pltpu.sync_copy(s1_hbm, s1t)
        pltpu.sync_copy(s2_hbm, s2t)
        pltpu.sync_copy(c_hbm, cvb)

        @functools.partial(plsc.parallel_loop, 0, (H * N) // 16, unroll=8)
        def _(i):
            den[pl.ds(i * 16, 16)] = jnp.zeros((16,), jnp.float32)

        for cidx in range(NCH):
            p = cidx % 2
            srcb, dstb, exb = srcbufs[p], dstbufs[p], exbufs[p]
            drain_in(cidx, p)
            if cidx + 2 < NCH:
                start_in(cidx + 2, p)
            if cidx >= 2:
                drain_out(cidx - 2, p)

            @functools.partial(plsc.parallel_loop, 0, CH // 16, unroll=4)
            def _(i):
                off = i * 16
                sv = srcb[pl.ds(off, 16)]
                dv = dstb[pl.ds(off, 16)]
                for h in range(H):
                    a1 = plsc.load_gather(s1t, [sv + h * N])
                    a2 = plsc.load_gather(s2t, [dv + h * N])
                    ex = _gelu_exp(a1 + a2, cvb[pl.ds(h * 16, 16)])
                    exb[pl.ds(h * CH + off, 16)] = ex

            def d_body(i, _):
                off = i * 16
                dv = dstb[pl.ds(off, 16)]
                for h in range(H):
                    ex = exb[pl.ds(h * CH + off, 16)]
                    plsc.addupdate_scatter(den, [dv + h * N], ex)
                return 0
            lax.fori_loop(0, CH // 16, d_body, 0)
            start_out(cidx, p)
        drain_out(NCH - 2, (NCH - 2) % 2)
        drain_out(NCH - 1, (NCH - 1) % 2)
        pltpu.sync_copy(den, den_hbm.at[pl.ds(wid * H * N, H * N)])

    return sc_a


def _make_sc_b(F, H):
    K = F // 4
    T = 32 // K
    ET = E // T
    CH = 4000
    NCH = ET // CH
    KH = K // H
    assert NCH % 2 == 0

    @functools.partial(
        pl.kernel,
        out_type=jax.ShapeDtypeStruct((T * F * N,), jnp.float32),
        mesh=_MESH,
        compiler_params=_SC_PARAMS,
        scratch_types=[pltpu.VMEM((4 * N,), jnp.float32),
                       pltpu.VMEM((4 * N,), jnp.float32),
                       pltpu.VMEM((CH,), jnp.int32),
                       pltpu.VMEM((CH,), jnp.int32),
                       pltpu.VMEM((CH,), jnp.int32),
                       pltpu.VMEM((CH,), jnp.int32),
                       pltpu.VMEM((CH,), jnp.float32),
                       pltpu.VMEM((CH,), jnp.float32),
                       pltpu.SemaphoreType.DMA,
                       pltpu.SemaphoreType.DMA],
    )
    def sc_b(src_hbm, dst_hbm, ex_hbm, z_hbm, num_hbm,
             zc, acc, srcb0, srcb1, dstb0, dstb1, exb0, exb1, sem0, sem1):
        cc = lax.axis_index("c")
        ss = lax.axis_index("s")
        wid = ss * 2 + cc
        chunk = wid // T
        part = wid % T
        f0 = chunk * 4
        head = chunk // KH
        srcbufs = (srcb0, srcb1)
        dstbufs = (dstb0, dstb1)
        exbufs = (exb0, exb1)
        sems = (sem0, sem1)
        ebase0 = part * ET

        def start(cidx, p):
            base = ebase0 + cidx * CH
            pltpu.async_copy(src_hbm.at[pl.ds(base, CH)], srcbufs[p], sems[p])
            pltpu.async_copy(dst_hbm.at[pl.ds(base, CH)], dstbufs[p], sems[p])
            pltpu.async_copy(ex_hbm.at[pl.ds(head * E + base, CH)],
                             exbufs[p], sems[p])

        def drain(cidx, p):
            base = ebase0 + cidx * CH
            pltpu.make_async_copy(src_hbm.at[pl.ds(base, CH)],
                                  srcbufs[p], sems[p]).wait()
            pltpu.make_async_copy(dst_hbm.at[pl.ds(base, CH)],
                                  dstbufs[p], sems[p]).wait()
            pltpu.make_async_copy(ex_hbm.at[pl.ds(head * E + base, CH)],
                                  exbufs[p], sems[p]).wait()

        start(0, 0)
        start(1, 1)
        pltpu.sync_copy(z_hbm.at[pl.ds(f0 * N, 4 * N)], zc)

        @functools.partial(plsc.parallel_loop, 0, (4 * N) // 16, unroll=8)
        def _(i):
            acc[pl.ds(i * 16, 16)] = jnp.zeros((16,), jnp.float32)

        def compute(p):
            srcb, dstb, exb = srcbufs[p], dstbufs[p], exbufs[p]

            @functools.partial(plsc.parallel_loop, 0, CH // 16, unroll=8)
            def _(i):
                off = i * 16
                sv = srcb[pl.ds(off, 16)]
                dv = dstb[pl.ds(off, 16)]
                w = exb[pl.ds(off, 16)]
                for r in range(4):
                    gth = plsc.load_gather(zc, [sv + r * N])
                    plsc.addupdate_scatter(acc, [dv + r * N], gth * w)

        def chunk_body(g, _):
            for p in range(2):
                cidx = g * 2 + p
                drain(cidx, p)
                compute(p)
                start(cidx + 2, p)
            return 0
        lax.fori_loop(0, NCH // 2 - 1, chunk_body, 0)
        for p in range(2):
            drain(NCH - 2 + p, p)
            compute(p)
        pltpu.sync_copy(acc, num_hbm.at[pl.ds((part * F + f0) * N, 4 * N)])

    return sc_b


def _attn_scalars(z, a1, a2):
    s1 = lax.dot_general(a1, z, (((1,), (0,)), ((), ())),
                         preferred_element_type=jnp.float32)
    s2 = lax.dot_general(a2, z, (((1,), (0,)), ((), ())),
                         preferred_element_type=jnp.float32)
    c = jnp.clip(jnp.max(s1) + jnp.max(s2), 0.0, 30.0)
    return s1, s2, c


def _merge(num_parts, den_parts, F, H, T):
    num = jnp.sum(num_parts, axis=0)
    den = jnp.sum(den_parts, axis=0)
    den = jnp.maximum(den, 1e-16)
    FH = F // H
    dens = [jnp.broadcast_to(den[h:h + 1], (FH, N)) for h in range(H)]
    return num / jnp.concatenate(dens, axis=0)


def _dense_call(body, out_shapes, *inputs):
    return pl.pallas_call(
        body,
        out_shape=out_shapes,
    )(*inputs)


def kernel(feature, params, edge_index):
    src = edge_index[0]
    dst = edge_index[1]

    def head_wb(p):
        return p["W"], p["b"], p["a"][:, 0]

    def d0_body(feat_ref, w1_ref, b1_ref, a1_ref, w2_ref, b2_ref, a2_ref,
                z_ref, s1_ref, s2_ref, c_ref):
        feat = feat_ref[...]
        for h, (w_ref, b_ref, a_ref) in enumerate(
                ((w1_ref, b1_ref, a1_ref), (w2_ref, b2_ref, a2_ref))):
            z = lax.dot_general(w_ref[...], feat, (((0,), (1,)), ((), ())),
                                preferred_element_type=jnp.float32)
            z = z + b_ref[...][:, None]
            a = a_ref[...]
            s1, s2, c = _attn_scalars(z, a[:1], a[1:])
            z_ref[pl.ds(h * 64, 64), :] = z
            s1_ref[pl.ds(h, 1), :] = s1
            s2_ref[pl.ds(h, 1), :] = s2
            c_ref[pl.ds(h, 1), :] = jnp.full((1, 16), c, jnp.float32)

    l0h = params["l0"]["heads"]
    w1, b1, av1 = head_wb(l0h[0])
    w2, b2, av2 = head_wb(l0h[1])
    a1m = jnp.stack([av1[:64], av1[64:]])
    a2m = jnp.stack([av2[:64], av2[64:]])
    zT0, s1_0, s2_0, c0 = _dense_call(
        d0_body,
        [jax.ShapeDtypeStruct((128, N), jnp.float32),
         jax.ShapeDtypeStruct((2, N), jnp.float32),
         jax.ShapeDtypeStruct((2, N), jnp.float32),
         jax.ShapeDtypeStruct((2, 16), jnp.float32)],
        feature, w1, b1, a1m, w2, b2, a2m)

    sc_a2 = _make_sc_a(2)
    sc_a1 = _make_sc_a(1)
    sc_b128 = _make_sc_b(128, 2)
    sc_b64 = _make_sc_b(64, 1)
    sc_b16 = _make_sc_b(16, 2)

    ex0, denp0 = sc_a2(src, dst, s1_0.reshape(-1), s2_0.reshape(-1),
                       c0.reshape(-1))
    nump0 = sc_b128(src, dst, ex0, zT0.reshape(-1))

    def mid_body(F_in, H_in, T_in, FH_out):
        def body(nump_ref, denp_ref, w_ref, b_ref, a_ref,
                 z_ref, s1_ref, s2_ref, c_ref):
            h_in = _merge(nump_ref[...], denp_ref[...], F_in, H_in, T_in)
            z = lax.dot_general(w_ref[...], h_in, (((0,), (0,)), ((), ())),
                                preferred_element_type=jnp.float32)
            z = z + b_ref[...][:, None]
            a = a_ref[...]
            s1, s2, c = _attn_scalars(z, a[:1], a[1:])
            z_ref[...] = z
            s1_ref[...] = s1
            s2_ref[...] = s2
            c_ref[...] = jnp.full((1, 16), c, jnp.float32)
        return body

    def mid2_body(F_in, H_in, T_in, FH_out):
        def body(nump_ref, denp_ref, w1_ref, b1_ref, a1_ref,
                 w2_ref, b2_ref, a2_ref, z_ref, s1_ref, s2_ref, c_ref):
            h_in = _merge(nump_ref[...], denp_ref[...], F_in, H_in, T_in)
            for h, (w_ref, b_ref, a_ref) in enumerate(
                    ((w1_ref, b1_ref, a1_ref), (w2_ref, b2_ref, a2_ref))):
                z = lax.dot_general(w_ref[...], h_in,
                                    (((0,), (0,)), ((), ())),
                                    preferred_element_type=jnp.float32)
                z = z + b_ref[...][:, None]
                a = a_ref[...]
                s1, s2, c = _attn_scalars(z, a[:1], a[1:])
                z_ref[pl.ds(h * FH_out, FH_out), :] = z
                s1_ref[pl.ds(h, 1), :] = s1
                s2_ref[pl.ds(h, 1), :] = s2
                c_ref[pl.ds(h, 1), :] = jnp.full((1, 16), c, jnp.float32)
        return body

    l0o = params["l0"]["out"]
    wo, bo, avo = head_wb(l0o)
    aom = jnp.stack([avo[:64], avo[64:]])
    zT1, s1_1, s2_1, c1 = _dense_call(
        mid_body(128, 2, 1, 64),
        [jax.ShapeDtypeStruct((64, N), jnp.float32),
         jax.ShapeDtypeStruct((1, N), jnp.float32),
         jax.ShapeDtypeStruct((1, N), jnp.float32),
         jax.ShapeDtypeStruct((1, 16), jnp.float32)],
        nump0.reshape(1, 128, N), denp0.reshape(32, 2, N), wo, bo, aom)

    ex1, denp1 = sc_a1(src, dst, s1_1.reshape(-1), s2_1.reshape(-1),
                       c1.reshape(-1))
    nump1 = sc_b64(src, dst, ex1, zT1.reshape(-1))

    l1h = params["l1"]["heads"]
    w1, b1, av1 = head_wb(l1h[0])
    w2, b2, av2 = head_wb(l1h[1])
    a1m = jnp.stack([av1[:64], av1[64:]])
    a2m = jnp.stack([av2[:64], av2[64:]])
    zT2, s1_2, s2_2, c2 = _dense_call(
        mid2_body(64, 1, 2, 64),
        [jax.ShapeDtypeStruct((128, N), jnp.float32),
         jax.ShapeDtypeStruct((2, N), jnp.float32),
         jax.ShapeDtypeStruct((2, N), jnp.float32),
         jax.ShapeDtypeStruct((2, 16), jnp.float32)],
        nump1.reshape(2, 64, N), denp1.reshape(32, 1, N),
        w1, b1, a1m, w2, b2, a2m)

    ex2, denp2 = sc_a2(src, dst, s1_2.reshape(-1), s2_2.reshape(-1),
                       c2.reshape(-1))
    nump2 = sc_b128(src, dst, ex2, zT2.reshape(-1))

    l1o = params["l1"]["out"]
    wo, bo, avo = head_wb(l1o)
    aom = jnp.stack([avo[:64], avo[64:]])
    zT3, s1_3, s2_3, c3 = _dense_call(
        mid_body(128, 2, 1, 64),
        [jax.ShapeDtypeStruct((64, N), jnp.float32),
         jax.ShapeDtypeStruct((1, N), jnp.float32),
         jax.ShapeDtypeStruct((1, N), jnp.float32),
         jax.ShapeDtypeStruct((1, 16), jnp.float32)],
        nump2.reshape(1, 128, N), denp2.reshape(32, 2, N), wo, bo, aom)

    ex3, denp3 = sc_a1(src, dst, s1_3.reshape(-1), s2_3.reshape(-1),
                       c3.reshape(-1))
    nump3 = sc_b64(src, dst, ex3, zT3.reshape(-1))

    def d4_body(nump_ref, denp_ref, w1_ref, b1_ref, a1_ref,
                w2_ref, b2_ref, a2_ref, z_ref, s1_ref, s2_ref, c_ref):
        h_in = _merge(nump_ref[...], denp_ref[...], 64, 1, 2)
        for h, (w_ref, b_ref, a_ref) in enumerate(
                ((w1_ref, b1_ref, a1_ref), (w2_ref, b2_ref, a2_ref))):
            z = lax.dot_general(w_ref[...], h_in, (((0,), (0,)), ((), ())),
                                preferred_element_type=jnp.float32)
            z = z + b_ref[...][:, None]
            a = a_ref[...]
            s1, s2, c = _attn_scalars(z, a[:1], a[1:])
            z_ref[pl.ds(h * 8, 7), :] = z
            z_ref[pl.ds(h * 8 + 7, 1), :] = jnp.zeros((1, N), jnp.float32)
            s1_ref[pl.ds(h, 1), :] = s1
            s2_ref[pl.ds(h, 1), :] = s2
            c_ref[pl.ds(h, 1), :] = jnp.full((1, 16), c, jnp.float32)

    olh = params["outl"]["heads"]
    w1, b1, av1 = head_wb(olh[0])
    w2, b2, av2 = head_wb(olh[1])
    a1m = jnp.stack([av1[:7], av1[7:]])
    a2m = jnp.stack([av2[:7], av2[7:]])
    zT4, s1_4, s2_4, c4 = _dense_call(
        d4_body,
        [jax.ShapeDtypeStruct((16, N), jnp.float32),
         jax.ShapeDtypeStruct((2, N), jnp.float32),
         jax.ShapeDtypeStruct((2, N), jnp.float32),
         jax.ShapeDtypeStruct((2, 16), jnp.float32)],
        nump3.reshape(2, 64, N), denp3.reshape(32, 1, N),
        w1, b1, a1m, w2, b2, a2m)

    ex4, denp4 = sc_a2(src, dst, s1_4.reshape(-1), s2_4.reshape(-1),
                       c4.reshape(-1))
    nump4 = sc_b16(src, dst, ex4, zT4.reshape(-1))

    def d5_body(nump_ref, denp_ref, lw_ref, lb_ref, out_ref):
        num = jnp.sum(nump_ref[...], axis=0)
        den = jnp.sum(denp_ref[...], axis=0)
        den = jnp.maximum(den, 1e-16)
        o0 = num[0:7] / den[0:1]
        o1 = num[8:15] / den[1:2]
        hmean = 0.5 * (o0 + o1)
        logits = lax.dot_general(hmean, lw_ref[...], (((0,), (0,)), ((), ())),
                                 preferred_element_type=jnp.float32)
        logits = logits + lb_ref[...][None, :]
        m = jnp.max(logits, axis=1, keepdims=True)
        ez = jnp.exp(logits - m)
        out_ref[...] = ez / jnp.sum(ez, axis=1, keepdims=True)

    lin = params["outl"]["lin"]
    out = _dense_call(
        d5_body,
        jax.ShapeDtypeStruct((N, 7), jnp.float32),
        nump4.reshape(8, 16, N), denp4.reshape(32, 2, N),
        lin["W"], lin["b"])
    return out


# speedup vs baseline: 86.1073x; 1.0994x over previous
import functools

import jax
import jax.numpy as jnp
from jax import lax
from jax.experimental import pallas as pl
from jax.experimental.pallas import tpu as pltpu
from jax.experimental.pallas import tpu_sc as plsc

N = 10000
E = 320000
_SC_PARAMS = pltpu.CompilerParams(needs_layout_passes=False)
_MESH = plsc.VectorSubcoreMesh(core_axis_name="c", subcore_axis_name="s")


def _gelu_exp(x, cvec):
    xa = jnp.abs(x) * 0.7071067811865476
    t = 1.0 / (1.0 + 0.3275911 * xa)
    poly = t * (0.254829592 + t * (-0.284496736 + t * (1.421413741
                + t * (-1.453152027 + t * 1.061405429))))
    erf = 1.0 - poly * jnp.exp(-xa * xa)
    erf = jnp.where(x >= 0, erf, -erf)
    g = 0.5 * x * (1.0 + erf)
    return jnp.exp(g - cvec)


def _make_sc_a(H):
    ET = E // 32
    CH = 2000
    NCH = ET // CH

    @functools.partial(
        pl.kernel,
        out_type=[jax.ShapeDtypeStruct((H * E,), jnp.float32),
                  jax.ShapeDtypeStruct((32 * H * N,), jnp.float32)],
        mesh=_MESH,
        compiler_params=_SC_PARAMS,
        scratch_types=[pltpu.VMEM((H * N,), jnp.float32),
                       pltpu.VMEM((H * N,), jnp.float32),
                       pltpu.VMEM((H * N,), jnp.float32),
                       pltpu.VMEM((CH,), jnp.int32),
                       pltpu.VMEM((CH,), jnp.int32),
                       pltpu.VMEM((H * CH,), jnp.float32),
                       pltpu.VMEM((H * CH,), jnp.float32),
                       pltpu.VMEM((H * 16,), jnp.float32),
                       pltpu.SemaphoreType.DMA,
                       pltpu.SemaphoreType.DMA,
                       pltpu.SemaphoreType.DMA,
                       pltpu.SemaphoreType.DMA],
    )
    def sc_a(sd_hbm, s1_hbm, s2_hbm, c_hbm, ex_hbm, den_hbm,
             s1t, s2t, den, sdb0, sdb1, exb0, exb1, cvb,
             semi0, semi1, semo0, semo1):
        cc = lax.axis_index("c")
        ss = lax.axis_index("s")
        wid = ss * 2 + cc
        sdbufs = (sdb0, sdb1)
        exbufs = (exb0, exb1)
        semis = (semi0, semi1)
        semos = (semo0, semo1)
        base0 = wid * ET

        def start_in(cidx, p):
            base = base0 + cidx * CH
            pltpu.async_copy(sd_hbm.at[pl.ds(base, CH)], sdbufs[p],
                             semis[p])

        def drain_in(cidx, p):
            base = base0 + cidx * CH
            pltpu.make_async_copy(sd_hbm.at[pl.ds(base, CH)], sdbufs[p],
                                  semis[p]).wait()

        def start_out(cidx, p):
            base = base0 + cidx * CH
            for h in range(H):
                pltpu.async_copy(exbufs[p].at[pl.ds(h * CH, CH)],
                                 ex_hbm.at[pl.ds(h * E + base, CH)],
                                 semos[p])

        def drain_out(cidx, p):
            base = base0 + cidx * CH
            for h in range(H):
                pltpu.make_async_copy(exbufs[p].at[pl.ds(h * CH, CH)],
                                      ex_hbm.at[pl.ds(h * E + base, CH)],
                                      semos[p]).wait()

        start_in(0, 0)
        start_in(1, 1)
        pltpu.sync_copy(s1_hbm, s1t)
        pltpu.sync_copy(s2_hbm, s2t)
        pltpu.sync_copy(c_hbm, cvb)

        @functools.partial(plsc.parallel_loop, 0, (H * N) // 16, unroll=8)
        def _(i):
            den[pl.ds(i * 16, 16)] = jnp.zeros((16,), jnp.float32)

        for cidx in range(NCH):
            p = cidx % 2
            sdb, exb = sdbufs[p], exbufs[p]
            drain_in(cidx, p)
            if cidx + 2 < NCH:
                start_in(cidx + 2, p)
            if cidx >= 2:
                drain_out(cidx - 2, p)

            @functools.partial(plsc.parallel_loop, 0, CH // 16, unroll=4)
            def _(i):
                off = i * 16
                sd = sdb[pl.ds(off, 16)]
                sv = jnp.bitwise_and(sd, 0xFFFF)
                dv = lax.shift_right_logical(sd, 16)
                for h in range(H):
                    a1 = plsc.load_gather(s1t, [sv + h * N])
                    a2 = plsc.load_gather(s2t, [dv + h * N])
                    ex = _gelu_exp(a1 + a2, cvb[pl.ds(h * 16, 16)])
                    exb[pl.ds(h * CH + off, 16)] = ex

            def d_body(i, _):
                off = i * 16
                dv = lax.shift_right_logical(sdb[pl.ds(off, 16)], 16)
                for h in range(H):
                    ex = exb[pl.ds(h * CH + off, 16)]
                    plsc.addupdate_scatter(den, [dv + h * N], ex)
                return 0
            lax.fori_loop(0, CH // 16, d_body, 0)
            start_out(cidx, p)
        drain_out(NCH - 2, (NCH - 2) % 2)
        drain_out(NCH - 1, (NCH - 1) % 2)
        pltpu.sync_copy(den, den_hbm.at[pl.ds(wid * H * N, H * N)])

    return sc_a


def _make_sc_b(F, H):
    K = F // 4
    T = 32 // K
    ET = E // T
    CH = 4000
    NCH = ET // CH
    KH = K // H
    assert NCH % 2 == 0

    @functools.partial(
        pl.kernel,
        out_type=jax.ShapeDtypeStruct((T * F * N,), jnp.float32),
        mesh=_MESH,
        compiler_params=_SC_PARAMS,
        scratch_types=[pltpu.VMEM((4 * N,), jnp.float32),
                       pltpu.VMEM((4 * N,), jnp.float32),
                       pltpu.VMEM((CH,), jnp.int32),
                       pltpu.VMEM((CH,), jnp.int32),
                       pltpu.VMEM((CH,), jnp.float32),
                       pltpu.VMEM((CH,), jnp.float32),
                       pltpu.SemaphoreType.DMA,
                       pltpu.SemaphoreType.DMA],
    )
    def sc_b(sd_hbm, ex_hbm, z_hbm, num_hbm,
             zc, acc, sdb0, sdb1, exb0, exb1, sem0, sem1):
        cc = lax.axis_index("c")
        ss = lax.axis_index("s")
        wid = ss * 2 + cc
        chunk = wid // T
        part = wid % T
        f0 = chunk * 4
        head = chunk // KH
        sdbufs = (sdb0, sdb1)
        exbufs = (exb0, exb1)
        sems = (sem0, sem1)
        ebase0 = part * ET

        def start(cidx, p):
            base = ebase0 + cidx * CH
            pltpu.async_copy(sd_hbm.at[pl.ds(base, CH)], sdbufs[p], sems[p])
            pltpu.async_copy(ex_hbm.at[pl.ds(head * E + base, CH)],
                             exbufs[p], sems[p])

        def drain(cidx, p):
            base = ebase0 + cidx * CH
            pltpu.make_async_copy(sd_hbm.at[pl.ds(base, CH)],
                                  sdbufs[p], sems[p]).wait()
            pltpu.make_async_copy(ex_hbm.at[pl.ds(head * E + base, CH)],
                                  exbufs[p], sems[p]).wait()

        start(0, 0)
        start(1, 1)
        pltpu.sync_copy(z_hbm.at[pl.ds(f0 * N, 4 * N)], zc)

        @functools.partial(plsc.parallel_loop, 0, (4 * N) // 16, unroll=8)
        def _(i):
            acc[pl.ds(i * 16, 16)] = jnp.zeros((16,), jnp.float32)

        def compute(p):
            sdb, exb = sdbufs[p], exbufs[p]

            @functools.partial(plsc.parallel_loop, 0, CH // 16, unroll=8)
            def _(i):
                off = i * 16
                sd = sdb[pl.ds(off, 16)]
                sv = jnp.bitwise_and(sd, 0xFFFF)
                dv = lax.shift_right_logical(sd, 16)
                w = exb[pl.ds(off, 16)]
                for r in range(4):
                    gth = plsc.load_gather(zc, [sv + r * N])
                    plsc.addupdate_scatter(acc, [dv + r * N], gth * w)

        def chunk_body(g, _):
            for p in range(2):
                cidx = g * 2 + p
                drain(cidx, p)
                compute(p)
                start(cidx + 2, p)
            return 0
        lax.fori_loop(0, NCH // 2 - 1, chunk_body, 0)
        for p in range(2):
            drain(NCH - 2 + p, p)
            compute(p)
        pltpu.sync_copy(acc, num_hbm.at[pl.ds((part * F + f0) * N, 4 * N)])

    return sc_b


def _attn_scalars(z, a1, a2):
    s1 = lax.dot_general(a1, z, (((1,), (0,)), ((), ())),
                         preferred_element_type=jnp.float32)
    s2 = lax.dot_general(a2, z, (((1,), (0,)), ((), ())),
                         preferred_element_type=jnp.float32)
    c = jnp.clip(jnp.max(s1) + jnp.max(s2), 0.0, 30.0)
    return s1, s2, c


def _merge(num_parts, den_parts, F, H, T):
    num = jnp.sum(num_parts, axis=0)
    den = jnp.sum(den_parts, axis=0)
    den = jnp.maximum(den, 1e-16)
    FH = F // H
    dens = [jnp.broadcast_to(den[h:h + 1], (FH, N)) for h in range(H)]
    return num / jnp.concatenate(dens, axis=0)


def _dense_call(body, out_shapes, *inputs):
    return pl.pallas_call(
        body,
        out_shape=out_shapes,
    )(*inputs)


def kernel(feature, params, edge_index):

    def head_wb(p):
        return p["W"], p["b"], p["a"][:, 0]

    def d0_body(feat_ref, ei_ref, w1_ref, b1_ref, a1_ref,
                w2_ref, b2_ref, a2_ref,
                z_ref, s1_ref, s2_ref, c_ref, sd_ref):
        ei = ei_ref[...]
        sd_ref[...] = jnp.bitwise_or(ei[0:1],
                                     lax.shift_left(ei[1:2], 16))
        feat = feat_ref[...]
        for h, (w_ref, b_ref, a_ref) in enumerate(
                ((w1_ref, b1_ref, a1_ref), (w2_ref, b2_ref, a2_ref))):
            z = lax.dot_general(w_ref[...], feat, (((0,), (1,)), ((), ())),
                                preferred_element_type=jnp.float32)
            z = z + b_ref[...][:, None]
            a = a_ref[...]
            s1, s2, c = _attn_scalars(z, a[:1], a[1:])
            z_ref[pl.ds(h * 64, 64), :] = z
            s1_ref[pl.ds(h, 1), :] = s1
            s2_ref[pl.ds(h, 1), :] = s2
            c_ref[pl.ds(h, 1), :] = jnp.full((1, 16), c, jnp.float32)

    l0h = params["l0"]["heads"]
    w1, b1, av1 = head_wb(l0h[0])
    w2, b2, av2 = head_wb(l0h[1])
    a1m = jnp.stack([av1[:64], av1[64:]])
    a2m = jnp.stack([av2[:64], av2[64:]])
    zT0, s1_0, s2_0, c0, sd2 = _dense_call(
        d0_body,
        [jax.ShapeDtypeStruct((128, N), jnp.float32),
         jax.ShapeDtypeStruct((2, N), jnp.float32),
         jax.ShapeDtypeStruct((2, N), jnp.float32),
         jax.ShapeDtypeStruct((2, 16), jnp.float32),
         jax.ShapeDtypeStruct((1, E), jnp.int32)],
        feature, edge_index, w1, b1, a1m, w2, b2, a2m)
    sd = sd2.reshape(-1)

    sc_a2 = _make_sc_a(2)
    sc_a1 = _make_sc_a(1)
    sc_b128 = _make_sc_b(128, 2)
    sc_b64 = _make_sc_b(64, 1)
    sc_b16 = _make_sc_b(16, 2)

    ex0, denp0 = sc_a2(sd, s1_0.reshape(-1), s2_0.reshape(-1),
                       c0.reshape(-1))
    nump0 = sc_b128(sd, ex0, zT0.reshape(-1))

    def mid_body(F_in, H_in, T_in, FH_out):
        def body(nump_ref, denp_ref, w_ref, b_ref, a_ref,
                 z_ref, s1_ref, s2_ref, c_ref):
            h_in = _merge(nump_ref[...], denp_ref[...], F_in, H_in, T_in)
            z = lax.dot_general(w_ref[...], h_in, (((0,), (0,)), ((), ())),
                                preferred_element_type=jnp.float32)
            z = z + b_ref[...][:, None]
            a = a_ref[...]
            s1, s2, c = _attn_scalars(z, a[:1], a[1:])
            z_ref[...] = z
            s1_ref[...] = s1
            s2_ref[...] = s2
            c_ref[...] = jnp.full((1, 16), c, jnp.float32)
        return body

    def mid2_body(F_in, H_in, T_in, FH_out):
        def body(nump_ref, denp_ref, w1_ref, b1_ref, a1_ref,
                 w2_ref, b2_ref, a2_ref, z_ref, s1_ref, s2_ref, c_ref):
            h_in = _merge(nump_ref[...], denp_ref[...], F_in, H_in, T_in)
            for h, (w_ref, b_ref, a_ref) in enumerate(
                    ((w1_ref, b1_ref, a1_ref), (w2_ref, b2_ref, a2_ref))):
                z = lax.dot_general(w_ref[...], h_in,
                                    (((0,), (0,)), ((), ())),
                                    preferred_element_type=jnp.float32)
                z = z + b_ref[...][:, None]
                a = a_ref[...]
                s1, s2, c = _attn_scalars(z, a[:1], a[1:])
                z_ref[pl.ds(h * FH_out, FH_out), :] = z
                s1_ref[pl.ds(h, 1), :] = s1
                s2_ref[pl.ds(h, 1), :] = s2
                c_ref[pl.ds(h, 1), :] = jnp.full((1, 16), c, jnp.float32)
        return body

    l0o = params["l0"]["out"]
    wo, bo, avo = head_wb(l0o)
    aom = jnp.stack([avo[:64], avo[64:]])
    zT1, s1_1, s2_1, c1 = _dense_call(
        mid_body(128, 2, 1, 64),
        [jax.ShapeDtypeStruct((64, N), jnp.float32),
         jax.ShapeDtypeStruct((1, N), jnp.float32),
         jax.ShapeDtypeStruct((1, N), jnp.float32),
         jax.ShapeDtypeStruct((1, 16), jnp.float32)],
        nump0.reshape(1, 128, N), denp0.reshape(32, 2, N), wo, bo, aom)

    ex1, denp1 = sc_a1(sd, s1_1.reshape(-1), s2_1.reshape(-1),
                       c1.reshape(-1))
    nump1 = sc_b64(sd, ex1, zT1.reshape(-1))

    l1h = params["l1"]["heads"]
    w1, b1, av1 = head_wb(l1h[0])
    w2, b2, av2 = head_wb(l1h[1])
    a1m = jnp.stack([av1[:64], av1[64:]])
    a2m = jnp.stack([av2[:64], av2[64:]])
    zT2, s1_2, s2_2, c2 = _dense_call(
        mid2_body(64, 1, 2, 64),
        [jax.ShapeDtypeStruct((128, N), jnp.float32),
         jax.ShapeDtypeStruct((2, N), jnp.float32),
         jax.ShapeDtypeStruct((2, N), jnp.float32),
         jax.ShapeDtypeStruct((2, 16), jnp.float32)],
        nump1.reshape(2, 64, N), denp1.reshape(32, 1, N),
        w1, b1, a1m, w2, b2, a2m)

    ex2, denp2 = sc_a2(sd, s1_2.reshape(-1), s2_2.reshape(-1),
                       c2.reshape(-1))
    nump2 = sc_b128(sd, ex2, zT2.reshape(-1))

    l1o = params["l1"]["out"]
    wo, bo, avo = head_wb(l1o)
    aom = jnp.stack([avo[:64], avo[64:]])
    zT3, s1_3, s2_3, c3 = _dense_call(
        mid_body(128, 2, 1, 64),
        [jax.ShapeDtypeStruct((64, N), jnp.float32),
         jax.ShapeDtypeStruct((1, N), jnp.float32),
         jax.ShapeDtypeStruct((1, N), jnp.float32),
         jax.ShapeDtypeStruct((1, 16), jnp.float32)],
        nump2.reshape(1, 128, N), denp2.reshape(32, 2, N), wo, bo, aom)

    ex3, denp3 = sc_a1(sd, s1_3.reshape(-1), s2_3.reshape(-1),
                       c3.reshape(-1))
    nump3 = sc_b64(sd, ex3, zT3.reshape(-1))

    def d4_body(nump_ref, denp_ref, w1_ref, b1_ref, a1_ref,
                w2_ref, b2_ref, a2_ref, z_ref, s1_ref, s2_ref, c_ref):
        h_in = _merge(nump_ref[...], denp_ref[...], 64, 1, 2)
        for h, (w_ref, b_ref, a_ref) in enumerate(
                ((w1_ref, b1_ref, a1_ref), (w2_ref, b2_ref, a2_ref))):
            z = lax.dot_general(w_ref[...], h_in, (((0,), (0,)), ((), ())),
                                preferred_element_type=jnp.float32)
            z = z + b_ref[...][:, None]
            a = a_ref[...]
            s1, s2, c = _attn_scalars(z, a[:1], a[1:])
            z_ref[pl.ds(h * 8, 7), :] = z
            z_ref[pl.ds(h * 8 + 7, 1), :] = jnp.zeros((1, N), jnp.float32)
            s1_ref[pl.ds(h, 1), :] = s1
            s2_ref[pl.ds(h, 1), :] = s2
            c_ref[pl.ds(h, 1), :] = jnp.full((1, 16), c, jnp.float32)

    olh = params["outl"]["heads"]
    w1, b1, av1 = head_wb(olh[0])
    w2, b2, av2 = head_wb(olh[1])
    a1m = jnp.stack([av1[:7], av1[7:]])
    a2m = jnp.stack([av2[:7], av2[7:]])
    zT4, s1_4, s2_4, c4 = _dense_call(
        d4_body,
        [jax.ShapeDtypeStruct((16, N), jnp.float32),
         jax.ShapeDtypeStruct((2, N), jnp.float32),
         jax.ShapeDtypeStruct((2, N), jnp.float32),
         jax.ShapeDtypeStruct((2, 16), jnp.float32)],
        nump3.reshape(2, 64, N), denp3.reshape(32, 1, N),
        w1, b1, a1m, w2, b2, a2m)

    ex4, denp4 = sc_a2(sd, s1_4.reshape(-1), s2_4.reshape(-1),
                       c4.reshape(-1))
    nump4 = sc_b16(sd, ex4, zT4.reshape(-1))

    def d5_body(nump_ref, denp_ref, lw_ref, lb_ref, out_ref):
        num = jnp.sum(nump_ref[...], axis=0)
        den = jnp.sum(denp_ref[...], axis=0)
        den = jnp.maximum(den, 1e-16)
        o0 = num[0:7] / den[0:1]
        o1 = num[8:15] / den[1:2]
        hmean = 0.5 * (o0 + o1)
        logits = lax.dot_general(hmean, lw_ref[...], (((0,), (0,)), ((), ())),
                                 preferred_element_type=jnp.float32)
        logits = logits + lb_ref[...][None, :]
        m = jnp.max(logits, axis=1, keepdims=True)
        ez = jnp.exp(logits - m)
        out_ref[...] = ez / jnp.sum(ez, axis=1, keepdims=True)

    lin = params["outl"]["lin"]
    out = _dense_call(
        d5_body,
        jax.ShapeDtypeStruct((N, 7), jnp.float32),
        nump4.reshape(8, 16, N), denp4.reshape(32, 2, N),
        lin["W"], lin["b"])
    return out


# speedup vs baseline: 89.7045x; 1.0418x over previous
import functools

import jax
import jax.numpy as jnp
from jax import lax
from jax.experimental import pallas as pl
from jax.experimental.pallas import tpu as pltpu
from jax.experimental.pallas import tpu_sc as plsc

N = 10000
E = 320000
_SC_PARAMS = pltpu.CompilerParams(needs_layout_passes=False)
_MESH = plsc.VectorSubcoreMesh(core_axis_name="c", subcore_axis_name="s")


def _gelu_exp(x, cvec):
    xa = jnp.abs(x) * 0.7071067811865476
    t = 1.0 / (1.0 + 0.3275911 * xa)
    poly = t * (0.254829592 + t * (-0.284496736 + t * (1.421413741
                + t * (-1.453152027 + t * 1.061405429))))
    erf = 1.0 - poly * jnp.exp(-xa * xa)
    erf = jnp.where(x >= 0, erf, -erf)
    g = 0.5 * x * (1.0 + erf)
    return jnp.exp(g - cvec)


def _make_sc_a(H):
    ET = E // 32
    CH = 2000
    NCH = ET // CH

    @functools.partial(
        pl.kernel,
        out_type=[jax.ShapeDtypeStruct((H * E,), jnp.float32),
                  jax.ShapeDtypeStruct((32 * H * N,), jnp.float32)],
        mesh=_MESH,
        compiler_params=_SC_PARAMS,
        scratch_types=[pltpu.VMEM((H * N,), jnp.float32),
                       pltpu.VMEM((H * N,), jnp.float32),
                       pltpu.VMEM((H * N,), jnp.float32),
                       pltpu.VMEM((CH,), jnp.int32),
                       pltpu.VMEM((CH,), jnp.int32),
                       pltpu.VMEM((H * CH,), jnp.float32),
                       pltpu.VMEM((H * CH,), jnp.float32),
                       pltpu.VMEM((H * 16,), jnp.float32),
                       pltpu.SemaphoreType.DMA,
                       pltpu.SemaphoreType.DMA,
                       pltpu.SemaphoreType.DMA,
                       pltpu.SemaphoreType.DMA],
    )
    def sc_a(sd_hbm, s1_hbm, s2_hbm, c_hbm, ex_hbm, den_hbm,
             s1t, s2t, den, sdb0, sdb1, exb0, exb1, cvb,
             semi0, semi1, semo0, semo1):
        cc = lax.axis_index("c")
        ss = lax.axis_index("s")
        wid = ss * 2 + cc
        sdbufs = (sdb0, sdb1)
        exbufs = (exb0, exb1)
        semis = (semi0, semi1)
        semos = (semo0, semo1)
        base0 = wid * ET

        def start_in(cidx, p):
            base = base0 + cidx * CH
            pltpu.async_copy(sd_hbm.at[pl.ds(base, CH)], sdbufs[p],
                             semis[p])

        def drain_in(cidx, p):
            base = base0 + cidx * CH
            pltpu.make_async_copy(sd_hbm.at[pl.ds(base, CH)], sdbufs[p],
                                  semis[p]).wait()

        def start_out(cidx, p):
            base = base0 + cidx * CH
            for h in range(H):
                pltpu.async_copy(exbufs[p].at[pl.ds(h * CH, CH)],
                                 ex_hbm.at[pl.ds(h * E + base, CH)],
                                 semos[p])

        def drain_out(cidx, p):
            base = base0 + cidx * CH
            for h in range(H):
                pltpu.make_async_copy(exbufs[p].at[pl.ds(h * CH, CH)],
                                      ex_hbm.at[pl.ds(h * E + base, CH)],
                                      semos[p]).wait()

        start_in(0, 0)
        start_in(1, 1)
        pltpu.sync_copy(s1_hbm, s1t)
        pltpu.sync_copy(s2_hbm, s2t)
        pltpu.sync_copy(c_hbm, cvb)

        @functools.partial(plsc.parallel_loop, 0, (H * N) // 16, unroll=8)
        def _(i):
            den[pl.ds(i * 16, 16)] = jnp.zeros((16,), jnp.float32)

        for cidx in range(NCH):
            p = cidx % 2
            sdb, exb = sdbufs[p], exbufs[p]
            drain_in(cidx, p)
            if cidx + 2 < NCH:
                start_in(cidx + 2, p)
            if cidx >= 2:
                drain_out(cidx - 2, p)

            @functools.partial(plsc.parallel_loop, 0, CH // 16, unroll=4)
            def _(i):
                off = i * 16
                sd = sdb[pl.ds(off, 16)]
                sv = jnp.bitwise_and(sd, 0xFFFF)
                dv = lax.shift_right_logical(sd, 16)
                for h in range(H):
                    a1 = plsc.load_gather(s1t, [sv + h * N])
                    a2 = plsc.load_gather(s2t, [dv + h * N])
                    ex = _gelu_exp(a1 + a2, cvb[pl.ds(h * 16, 16)])
                    exb[pl.ds(h * CH + off, 16)] = ex
                    plsc.addupdate_scatter(den, [dv + h * N], ex)

            start_out(cidx, p)
        drain_out(NCH - 2, (NCH - 2) % 2)
        drain_out(NCH - 1, (NCH - 1) % 2)
        pltpu.sync_copy(den, den_hbm.at[pl.ds(wid * H * N, H * N)])

    return sc_a


def _make_sc_b(F, H):
    K = F // 4
    T = 32 // K
    ET = E // T
    CH = 4000
    NCH = ET // CH
    KH = K // H
    assert NCH % 2 == 0

    @functools.partial(
        pl.kernel,
        out_type=jax.ShapeDtypeStruct((T * F * N,), jnp.float32),
        mesh=_MESH,
        compiler_params=_SC_PARAMS,
        scratch_types=[pltpu.VMEM((4 * N,), jnp.float32),
                       pltpu.VMEM((4 * N,), jnp.float32),
                       pltpu.VMEM((CH,), jnp.int32),
                       pltpu.VMEM((CH,), jnp.int32),
                       pltpu.VMEM((CH,), jnp.float32),
                       pltpu.VMEM((CH,), jnp.float32),
                       pltpu.SemaphoreType.DMA,
                       pltpu.SemaphoreType.DMA],
    )
    def sc_b(sd_hbm, ex_hbm, z_hbm, num_hbm,
             zc, acc, sdb0, sdb1, exb0, exb1, sem0, sem1):
        cc = lax.axis_index("c")
        ss = lax.axis_index("s")
        wid = ss * 2 + cc
        chunk = wid // T
        part = wid % T
        f0 = chunk * 4
        head = chunk // KH
        sdbufs = (sdb0, sdb1)
        exbufs = (exb0, exb1)
        sems = (sem0, sem1)
        ebase0 = part * ET

        def start(cidx, p):
            base = ebase0 + cidx * CH
            pltpu.async_copy(sd_hbm.at[pl.ds(base, CH)], sdbufs[p], sems[p])
            pltpu.async_copy(ex_hbm.at[pl.ds(head * E + base, CH)],
                             exbufs[p], sems[p])

        def drain(cidx, p):
            base = ebase0 + cidx * CH
            pltpu.make_async_copy(sd_hbm.at[pl.ds(base, CH)],
                                  sdbufs[p], sems[p]).wait()
            pltpu.make_async_copy(ex_hbm.at[pl.ds(head * E + base, CH)],
                                  exbufs[p], sems[p]).wait()

        start(0, 0)
        start(1, 1)
        pltpu.sync_copy(z_hbm.at[pl.ds(f0 * N, 4 * N)], zc)

        @functools.partial(plsc.parallel_loop, 0, (4 * N) // 16, unroll=8)
        def _(i):
            acc[pl.ds(i * 16, 16)] = jnp.zeros((16,), jnp.float32)

        def compute(p):
            sdb, exb = sdbufs[p], exbufs[p]

            @functools.partial(plsc.parallel_loop, 0, CH // 16, unroll=8)
            def _(i):
                off = i * 16
                sd = sdb[pl.ds(off, 16)]
                sv = jnp.bitwise_and(sd, 0xFFFF)
                dv = lax.shift_right_logical(sd, 16)
                w = exb[pl.ds(off, 16)]
                for r in range(4):
                    gth = plsc.load_gather(zc, [sv + r * N])
                    plsc.addupdate_scatter(acc, [dv + r * N], gth * w)

        def chunk_body(g, _):
            for p in range(2):
                cidx = g * 2 + p
                drain(cidx, p)
                compute(p)
                start(cidx + 2, p)
            return 0
        lax.fori_loop(0, NCH // 2 - 1, chunk_body, 0)
        for p in range(2):
            drain(NCH - 2 + p, p)
            compute(p)
        pltpu.sync_copy(acc, num_hbm.at[pl.ds((part * F + f0) * N, 4 * N)])

    return sc_b


def _attn_scalars(z, a1, a2):
    s1 = lax.dot_general(a1, z, (((1,), (0,)), ((), ())),
                         preferred_element_type=jnp.float32)
    s2 = lax.dot_general(a2, z, (((1,), (0,)), ((), ())),
                         preferred_element_type=jnp.float32)
    c = jnp.clip(jnp.max(s1) + jnp.max(s2), 0.0, 30.0)
    return s1, s2, c


def _merge(num_parts, den_parts, F, H, T):
    num = jnp.sum(num_parts, axis=0)
    den = jnp.sum(den_parts, axis=0)
    den = jnp.maximum(den, 1e-16)
    FH = F // H
    dens = [jnp.broadcast_to(den[h:h + 1], (FH, N)) for h in range(H)]
    return num / jnp.concatenate(dens, axis=0)


def _dense_call(body, out_shapes, *inputs):
    return pl.pallas_call(
        body,
        out_shape=out_shapes,
    )(*inputs)


def kernel(feature, params, edge_index):

    def head_wb(p):
        return p["W"], p["b"], p["a"][:, 0]

    def d0_body(feat_ref, ei_ref, w1_ref, b1_ref, a1_ref,
                w2_ref, b2_ref, a2_ref,
                z_ref, s1_ref, s2_ref, c_ref, sd_ref):
        ei = ei_ref[...]
        sd_ref[...] = jnp.bitwise_or(ei[0:1],
                                     lax.shift_left(ei[1:2], 16))
        feat = feat_ref[...]
        for h, (w_ref, b_ref, a_ref) in enumerate(
                ((w1_ref, b1_ref, a1_ref), (w2_ref, b2_ref, a2_ref))):
            z = lax.dot_general(w_ref[...], feat, (((0,), (1,)), ((), ())),
                                preferred_element_type=jnp.float32)
            z = z + b_ref[...][:, None]
            a = a_ref[...]
            s1, s2, c = _attn_scalars(z, a[:1], a[1:])
            z_ref[pl.ds(h * 64, 64), :] = z
            s1_ref[pl.ds(h, 1), :] = s1
            s2_ref[pl.ds(h, 1), :] = s2
            c_ref[pl.ds(h, 1), :] = jnp.full((1, 16), c, jnp.float32)

    l0h = params["l0"]["heads"]
    w1, b1, av1 = head_wb(l0h[0])
    w2, b2, av2 = head_wb(l0h[1])
    a1m = jnp.stack([av1[:64], av1[64:]])
    a2m = jnp.stack([av2[:64], av2[64:]])
    zT0, s1_0, s2_0, c0, sd2 = _dense_call(
        d0_body,
        [jax.ShapeDtypeStruct((128, N), jnp.float32),
         jax.ShapeDtypeStruct((2, N), jnp.float32),
         jax.ShapeDtypeStruct((2, N), jnp.float32),
         jax.ShapeDtypeStruct((2, 16), jnp.float32),
         jax.ShapeDtypeStruct((1, E), jnp.int32)],
        feature, edge_index, w1, b1, a1m, w2, b2, a2m)
    sd = sd2.reshape(-1)

    sc_a2 = _make_sc_a(2)
    sc_a1 = _make_sc_a(1)
    sc_b128 = _make_sc_b(128, 2)
    sc_b64 = _make_sc_b(64, 1)
    sc_b16 = _make_sc_b(16, 2)

    ex0, denp0 = sc_a2(sd, s1_0.reshape(-1), s2_0.reshape(-1),
                       c0.reshape(-1))
    nump0 = sc_b128(sd, ex0, zT0.reshape(-1))

    def mid_body(F_in, H_in, T_in, FH_out):
        def body(nump_ref, denp_ref, w_ref, b_ref, a_ref,
                 z_ref, s1_ref, s2_ref, c_ref):
            h_in = _merge(nump_ref[...], denp_ref[...], F_in, H_in, T_in)
            z = lax.dot_general(w_ref[...], h_in, (((0,), (0,)), ((), ())),
                                preferred_element_type=jnp.float32)
            z = z + b_ref[...][:, None]
            a = a_ref[...]
            s1, s2, c = _attn_scalars(z, a[:1], a[1:])
            z_ref[...] = z
            s1_ref[...] = s1
            s2_ref[...] = s2
            c_ref[...] = jnp.full((1, 16), c, jnp.float32)
        return body

    def mid2_body(F_in, H_in, T_in, FH_out):
        def body(nump_ref, denp_ref, w1_ref, b1_ref, a1_ref,
                 w2_ref, b2_ref, a2_ref, z_ref, s1_ref, s2_ref, c_ref):
            h_in = _merge(nump_ref[...], denp_ref[...], F_in, H_in, T_in)
            for h, (w_ref, b_ref, a_ref) in enumerate(
                    ((w1_ref, b1_ref, a1_ref), (w2_ref, b2_ref, a2_ref))):
                z = lax.dot_general(w_ref[...], h_in,
                                    (((0,), (0,)), ((), ())),
                                    preferred_element_type=jnp.float32)
                z = z + b_ref[...][:, None]
                a = a_ref[...]
                s1, s2, c = _attn_scalars(z, a[:1], a[1:])
                z_ref[pl.ds(h * FH_out, FH_out), :] = z
                s1_ref[pl.ds(h, 1), :] = s1
                s2_ref[pl.ds(h, 1), :] = s2
                c_ref[pl.ds(h, 1), :] = jnp.full((1, 16), c, jnp.float32)
        return body

    l0o = params["l0"]["out"]
    wo, bo, avo = head_wb(l0o)
    aom = jnp.stack([avo[:64], avo[64:]])
    zT1, s1_1, s2_1, c1 = _dense_call(
        mid_body(128, 2, 1, 64),
        [jax.ShapeDtypeStruct((64, N), jnp.float32),
         jax.ShapeDtypeStruct((1, N), jnp.float32),
         jax.ShapeDtypeStruct((1, N), jnp.float32),
         jax.ShapeDtypeStruct((1, 16), jnp.float32)],
        nump0.reshape(1, 128, N), denp0.reshape(32, 2, N), wo, bo, aom)

    ex1, denp1 = sc_a1(sd, s1_1.reshape(-1), s2_1.reshape(-1),
                       c1.reshape(-1))
    nump1 = sc_b64(sd, ex1, zT1.reshape(-1))

    l1h = params["l1"]["heads"]
    w1, b1, av1 = head_wb(l1h[0])
    w2, b2, av2 = head_wb(l1h[1])
    a1m = jnp.stack([av1[:64], av1[64:]])
    a2m = jnp.stack([av2[:64], av2[64:]])
    zT2, s1_2, s2_2, c2 = _dense_call(
        mid2_body(64, 1, 2, 64),
        [jax.ShapeDtypeStruct((128, N), jnp.float32),
         jax.ShapeDtypeStruct((2, N), jnp.float32),
         jax.ShapeDtypeStruct((2, N), jnp.float32),
         jax.ShapeDtypeStruct((2, 16), jnp.float32)],
        nump1.reshape(2, 64, N), denp1.reshape(32, 1, N),
        w1, b1, a1m, w2, b2, a2m)

    ex2, denp2 = sc_a2(sd, s1_2.reshape(-1), s2_2.reshape(-1),
                       c2.reshape(-1))
    nump2 = sc_b128(sd, ex2, zT2.reshape(-1))

    l1o = params["l1"]["out"]
    wo, bo, avo = head_wb(l1o)
    aom = jnp.stack([avo[:64], avo[64:]])
    zT3, s1_3, s2_3, c3 = _dense_call(
        mid_body(128, 2, 1, 64),
        [jax.ShapeDtypeStruct((64, N), jnp.float32),
         jax.ShapeDtypeStruct((1, N), jnp.float32),
         jax.ShapeDtypeStruct((1, N), jnp.float32),
         jax.ShapeDtypeStruct((1, 16), jnp.float32)],
        nump2.reshape(1, 128, N), denp2.reshape(32, 2, N), wo, bo, aom)

    ex3, denp3 = sc_a1(sd, s1_3.reshape(-1), s2_3.reshape(-1),
                       c3.reshape(-1))
    nump3 = sc_b64(sd, ex3, zT3.reshape(-1))

    def d4_body(nump_ref, denp_ref, w1_ref, b1_ref, a1_ref,
                w2_ref, b2_ref, a2_ref, z_ref, s1_ref, s2_ref, c_ref):
        h_in = _merge(nump_ref[...], denp_ref[...], 64, 1, 2)
        for h, (w_ref, b_ref, a_ref) in enumerate(
                ((w1_ref, b1_ref, a1_ref), (w2_ref, b2_ref, a2_ref))):
            z = lax.dot_general(w_ref[...], h_in, (((0,), (0,)), ((), ())),
                                preferred_element_type=jnp.float32)
            z = z + b_ref[...][:, None]
            a = a_ref[...]
            s1, s2, c = _attn_scalars(z, a[:1], a[1:])
            z_ref[pl.ds(h * 8, 7), :] = z
            z_ref[pl.ds(h * 8 + 7, 1), :] = jnp.zeros((1, N), jnp.float32)
            s1_ref[pl.ds(h, 1), :] = s1
            s2_ref[pl.ds(h, 1), :] = s2
            c_ref[pl.ds(h, 1), :] = jnp.full((1, 16), c, jnp.float32)

    olh = params["outl"]["heads"]
    w1, b1, av1 = head_wb(olh[0])
    w2, b2, av2 = head_wb(olh[1])
    a1m = jnp.stack([av1[:7], av1[7:]])
    a2m = jnp.stack([av2[:7], av2[7:]])
    zT4, s1_4, s2_4, c4 = _dense_call(
        d4_body,
        [jax.ShapeDtypeStruct((16, N), jnp.float32),
         jax.ShapeDtypeStruct((2, N), jnp.float32),
         jax.ShapeDtypeStruct((2, N), jnp.float32),
         jax.ShapeDtypeStruct((2, 16), jnp.float32)],
        nump3.reshape(2, 64, N), denp3.reshape(32, 1, N),
        w1, b1, a1m, w2, b2, a2m)

    ex4, denp4 = sc_a2(sd, s1_4.reshape(-1), s2_4.reshape(-1),
                       c4.reshape(-1))
    nump4 = sc_b16(sd, ex4, zT4.reshape(-1))

    def d5_body(nump_ref, denp_ref, lw_ref, lb_ref, out_ref):
        num = jnp.sum(nump_ref[...], axis=0)
        den = jnp.sum(denp_ref[...], axis=0)
        den = jnp.maximum(den, 1e-16)
        o0 = num[0:7] / den[0:1]
        o1 = num[8:15] / den[1:2]
        hmean = 0.5 * (o0 + o1)
        logits = lax.dot_general(hmean, lw_ref[...], (((0,), (0,)), ((), ())),
                                 preferred_element_type=jnp.float32)
        logits = logits + lb_ref[...][None, :]
        m = jnp.max(logits, axis=1, keepdims=True)
        ez = jnp.exp(logits - m)
        out_ref[...] = ez / jnp.sum(ez, axis=1, keepdims=True)

    lin = params["outl"]["lin"]
    out = _dense_call(
        d5_body,
        jax.ShapeDtypeStruct((N, 7), jnp.float32),
        nump4.reshape(8, 16, N), denp4.reshape(32, 2, N),
        lin["W"], lin["b"])
    return out


# speedup vs baseline: 94.5646x; 1.0542x over previous
import functools

import jax
import jax.numpy as jnp
from jax import lax
from jax.experimental import pallas as pl
from jax.experimental.pallas import tpu as pltpu
from jax.experimental.pallas import tpu_sc as plsc

N = 10000
E = 320000
_SC_PARAMS = pltpu.CompilerParams(needs_layout_passes=False)
_MESH = plsc.VectorSubcoreMesh(core_axis_name="c", subcore_axis_name="s")


def _gelu_exp(x, cvec):
    xa = jnp.abs(x) * 0.7071067811865476
    t = 1.0 / (1.0 + 0.3275911 * xa)
    poly = t * (0.254829592 + t * (-0.284496736 + t * (1.421413741
                + t * (-1.453152027 + t * 1.061405429))))
    erf = 1.0 - poly * jnp.exp(-xa * xa)
    erf = jnp.where(x >= 0, erf, -erf)
    g = 0.5 * x * (1.0 + erf)
    return jnp.exp(g - cvec)


def _make_sc_a(H):
    ET = E // 32
    CH = 2000
    NCH = ET // CH

    @functools.partial(
        pl.kernel,
        out_type=[jax.ShapeDtypeStruct((H * E,), jnp.float32),
                  jax.ShapeDtypeStruct((32 * H * N,), jnp.float32)],
        mesh=_MESH,
        compiler_params=_SC_PARAMS,
        scratch_types=[pltpu.VMEM((H * N,), jnp.float32),
                       pltpu.VMEM((H * N,), jnp.float32),
                       pltpu.VMEM((H * N,), jnp.float32),
                       pltpu.VMEM((CH,), jnp.int32),
                       pltpu.VMEM((CH,), jnp.int32),
                       pltpu.VMEM((H * CH,), jnp.float32),
                       pltpu.VMEM((H * CH,), jnp.float32),
                       pltpu.VMEM((H * 16,), jnp.float32),
                       pltpu.SemaphoreType.DMA,
                       pltpu.SemaphoreType.DMA,
                       pltpu.SemaphoreType.DMA,
                       pltpu.SemaphoreType.DMA],
    )
    def sc_a(sd_hbm, s1_hbm, s2_hbm, c_hbm, ex_hbm, den_hbm,
             s1t, s2t, den, sdb0, sdb1, exb0, exb1, cvb,
             semi0, semi1, semo0, semo1):
        cc = lax.axis_index("c")
        ss = lax.axis_index("s")
        wid = ss * 2 + cc
        sdbufs = (sdb0, sdb1)
        exbufs = (exb0, exb1)
        semis = (semi0, semi1)
        semos = (semo0, semo1)
        base0 = wid * ET

        def start_in(cidx, p):
            base = base0 + cidx * CH
            pltpu.async_copy(sd_hbm.at[pl.ds(base, CH)], sdbufs[p],
                             semis[p])

        def drain_in(cidx, p):
            base = base0 + cidx * CH
            pltpu.make_async_copy(sd_hbm.at[pl.ds(base, CH)], sdbufs[p],
                                  semis[p]).wait()

        def start_out(cidx, p):
            base = base0 + cidx * CH
            for h in range(H):
                pltpu.async_copy(exbufs[p].at[pl.ds(h * CH, CH)],
                                 ex_hbm.at[pl.ds(h * E + base, CH)],
                                 semos[p])

        def drain_out(cidx, p):
            base = base0 + cidx * CH
            for h in range(H):
                pltpu.make_async_copy(exbufs[p].at[pl.ds(h * CH, CH)],
                                      ex_hbm.at[pl.ds(h * E + base, CH)],
                                      semos[p]).wait()

        start_in(0, 0)
        start_in(1, 1)
        pltpu.sync_copy(s1_hbm, s1t)
        pltpu.sync_copy(s2_hbm, s2t)
        pltpu.sync_copy(c_hbm, cvb)

        @functools.partial(plsc.parallel_loop, 0, (H * N) // 16, unroll=8)
        def _(i):
            den[pl.ds(i * 16, 16)] = jnp.zeros((16,), jnp.float32)

        for cidx in range(NCH):
            p = cidx % 2
            sdb, exb = sdbufs[p], exbufs[p]
            drain_in(cidx, p)
            if cidx + 2 < NCH:
                start_in(cidx + 2, p)
            if cidx >= 2:
                drain_out(cidx - 2, p)

            @functools.partial(plsc.parallel_loop, 0, CH // 16, unroll=4)
            def _(i):
                off = i * 16
                sd = sdb[pl.ds(off, 16)]
                sv = jnp.bitwise_and(sd, 0xFFFF)
                dv = lax.shift_right_logical(sd, 16)
                for h in range(H):
                    a1 = plsc.load_gather(s1t, [sv + h * N])
                    a2 = plsc.load_gather(s2t, [dv + h * N])
                    ex = _gelu_exp(a1 + a2, cvb[pl.ds(h * 16, 16)])
                    exb[pl.ds(h * CH + off, 16)] = ex
                    plsc.addupdate_scatter(den, [dv + h * N], ex)

            start_out(cidx, p)
        drain_out(NCH - 2, (NCH - 2) % 2)
        drain_out(NCH - 1, (NCH - 1) % 2)
        pltpu.sync_copy(den, den_hbm.at[pl.ds(wid * H * N, H * N)])

    return sc_a


def _make_sc_b(F, H):
    K = F // 4
    T = 32 // K
    ET = E // T
    CH = 8000 if T < 8 else 4000
    NCH = ET // CH
    KH = K // H
    assert NCH % 2 == 0

    @functools.partial(
        pl.kernel,
        out_type=jax.ShapeDtypeStruct((T * F * N,), jnp.float32),
        mesh=_MESH,
        compiler_params=_SC_PARAMS,
        scratch_types=[pltpu.VMEM((4 * N,), jnp.float32),
                       pltpu.VMEM((4 * N,), jnp.float32),
                       pltpu.VMEM((CH,), jnp.int32),
                       pltpu.VMEM((CH,), jnp.int32),
                       pltpu.VMEM((CH,), jnp.float32),
                       pltpu.VMEM((CH,), jnp.float32),
                       pltpu.SemaphoreType.DMA,
                       pltpu.SemaphoreType.DMA],
    )
    def sc_b(sd_hbm, ex_hbm, z_hbm, num_hbm,
             zc, acc, sdb0, sdb1, exb0, exb1, sem0, sem1):
        cc = lax.axis_index("c")
        ss = lax.axis_index("s")
        wid = ss * 2 + cc
        chunk = wid // T
        part = wid % T
        f0 = chunk * 4
        head = chunk // KH
        sdbufs = (sdb0, sdb1)
        exbufs = (exb0, exb1)
        sems = (sem0, sem1)
        ebase0 = part * ET

        def start(cidx, p):
            base = ebase0 + cidx * CH
            pltpu.async_copy(sd_hbm.at[pl.ds(base, CH)], sdbufs[p], sems[p])
            pltpu.async_copy(ex_hbm.at[pl.ds(head * E + base, CH)],
                             exbufs[p], sems[p])

        def drain(cidx, p):
            base = ebase0 + cidx * CH
            pltpu.make_async_copy(sd_hbm.at[pl.ds(base, CH)],
                                  sdbufs[p], sems[p]).wait()
            pltpu.make_async_copy(ex_hbm.at[pl.ds(head * E + base, CH)],
                                  exbufs[p], sems[p]).wait()

        start(0, 0)
        start(1, 1)
        pltpu.sync_copy(z_hbm.at[pl.ds(f0 * N, 4 * N)], zc)

        @functools.partial(plsc.parallel_loop, 0, (4 * N) // 16, unroll=8)
        def _(i):
            acc[pl.ds(i * 16, 16)] = jnp.zeros((16,), jnp.float32)

        def compute(p):
            sdb, exb = sdbufs[p], exbufs[p]

            @functools.partial(plsc.parallel_loop, 0, CH // 16, unroll=8)
            def _(i):
                off = i * 16
                sd = sdb[pl.ds(off, 16)]
                sv = jnp.bitwise_and(sd, 0xFFFF)
                dv = lax.shift_right_logical(sd, 16)
                w = exb[pl.ds(off, 16)]
                for r in range(4):
                    gth = plsc.load_gather(zc, [sv + r * N])
                    plsc.addupdate_scatter(acc, [dv + r * N], gth * w)

        def chunk_body(g, _):
            for p in range(2):
                cidx = g * 2 + p
                drain(cidx, p)
                compute(p)
                start(cidx + 2, p)
            return 0
        lax.fori_loop(0, NCH // 2 - 1, chunk_body, 0)
        for p in range(2):
            drain(NCH - 2 + p, p)
            compute(p)
        pltpu.sync_copy(acc, num_hbm.at[pl.ds((part * F + f0) * N, 4 * N)])

    return sc_b


def _attn_scalars(z, a1, a2):
    s1 = lax.dot_general(a1, z, (((1,), (0,)), ((), ())),
                         preferred_element_type=jnp.float32)
    s2 = lax.dot_general(a2, z, (((1,), (0,)), ((), ())),
                         preferred_element_type=jnp.float32)
    c = jnp.clip(jnp.max(s1) + jnp.max(s2), 0.0, 30.0)
    return s1, s2, c


def _merge(num_parts, den_parts, F, H, T):
    num = jnp.sum(num_parts, axis=0)
    den = jnp.sum(den_parts, axis=0)
    den = jnp.maximum(den, 1e-16)
    FH = F // H
    dens = [jnp.broadcast_to(den[h:h + 1], (FH, N)) for h in range(H)]
    return num / jnp.concatenate(dens, axis=0)


def _dense_call(body, out_shapes, *inputs):
    return pl.pallas_call(
        body,
        out_shape=out_shapes,
    )(*inputs)


def kernel(feature, params, edge_index):

    def head_wb(p):
        return p["W"], p["b"], p["a"][:, 0]

    def d0_body(feat_ref, ei_ref, w1_ref, b1_ref, a1_ref,
                w2_ref, b2_ref, a2_ref,
                z_ref, s1_ref, s2_ref, c_ref, sd_ref):
        ei = ei_ref[...]
        sd_ref[...] = jnp.bitwise_or(ei[0:1],
                                     lax.shift_left(ei[1:2], 16))
        feat = feat_ref[...]
        for h, (w_ref, b_ref, a_ref) in enumerate(
                ((w1_ref, b1_ref, a1_ref), (w2_ref, b2_ref, a2_ref))):
            z = lax.dot_general(w_ref[...], feat, (((0,), (1,)), ((), ())),
                                preferred_element_type=jnp.float32)
            z = z + b_ref[...][:, None]
            a = a_ref[...]
            s1, s2, c = _attn_scalars(z, a[:1], a[1:])
            z_ref[pl.ds(h * 64, 64), :] = z
            s1_ref[pl.ds(h, 1), :] = s1
            s2_ref[pl.ds(h, 1), :] = s2
            c_ref[pl.ds(h, 1), :] = jnp.full((1, 16), c, jnp.float32)

    l0h = params["l0"]["heads"]
    w1, b1, av1 = head_wb(l0h[0])
    w2, b2, av2 = head_wb(l0h[1])
    a1m = jnp.stack([av1[:64], av1[64:]])
    a2m = jnp.stack([av2[:64], av2[64:]])
    zT0, s1_0, s2_0, c0, sd2 = _dense_call(
        d0_body,
        [jax.ShapeDtypeStruct((128, N), jnp.float32),
         jax.ShapeDtypeStruct((2, N), jnp.float32),
         jax.ShapeDtypeStruct((2, N), jnp.float32),
         jax.ShapeDtypeStruct((2, 16), jnp.float32),
         jax.ShapeDtypeStruct((1, E), jnp.int32)],
        feature, edge_index, w1, b1, a1m, w2, b2, a2m)
    sd = sd2.reshape(-1)

    sc_a2 = _make_sc_a(2)
    sc_a1 = _make_sc_a(1)
    sc_b128 = _make_sc_b(128, 2)
    sc_b64 = _make_sc_b(64, 1)
    sc_b16 = _make_sc_b(16, 2)

    ex0, denp0 = sc_a2(sd, s1_0.reshape(-1), s2_0.reshape(-1),
                       c0.reshape(-1))
    nump0 = sc_b128(sd, ex0, zT0.reshape(-1))

    def mid_body(F_in, H_in, T_in, FH_out):
        def body(nump_ref, denp_ref, w_ref, b_ref, a_ref,
                 z_ref, s1_ref, s2_ref, c_ref):
            h_in = _merge(nump_ref[...], denp_ref[...], F_in, H_in, T_in)
            z = lax.dot_general(w_ref[...], h_in, (((0,), (0,)), ((), ())),
                                preferred_element_type=jnp.float32)
            z = z + b_ref[...][:, None]
            a = a_ref[...]
            s1, s2, c = _attn_scalars(z, a[:1], a[1:])
            z_ref[...] = z
            s1_ref[...] = s1
            s2_ref[...] = s2
            c_ref[...] = jnp.full((1, 16), c, jnp.float32)
        return body

    def mid2_body(F_in, H_in, T_in, FH_out):
        def body(nump_ref, denp_ref, w1_ref, b1_ref, a1_ref,
                 w2_ref, b2_ref, a2_ref, z_ref, s1_ref, s2_ref, c_ref):
            h_in = _merge(nump_ref[...], denp_ref[...], F_in, H_in, T_in)
            for h, (w_ref, b_ref, a_ref) in enumerate(
                    ((w1_ref, b1_ref, a1_ref), (w2_ref, b2_ref, a2_ref))):
                z = lax.dot_general(w_ref[...], h_in,
                                    (((0,), (0,)), ((), ())),
                                    preferred_element_type=jnp.float32)
                z = z + b_ref[...][:, None]
                a = a_ref[...]
                s1, s2, c = _attn_scalars(z, a[:1], a[1:])
                z_ref[pl.ds(h * FH_out, FH_out), :] = z
                s1_ref[pl.ds(h, 1), :] = s1
                s2_ref[pl.ds(h, 1), :] = s2
                c_ref[pl.ds(h, 1), :] = jnp.full((1, 16), c, jnp.float32)
        return body

    l0o = params["l0"]["out"]
    wo, bo, avo = head_wb(l0o)
    aom = jnp.stack([avo[:64], avo[64:]])
    zT1, s1_1, s2_1, c1 = _dense_call(
        mid_body(128, 2, 1, 64),
        [jax.ShapeDtypeStruct((64, N), jnp.float32),
         jax.ShapeDtypeStruct((1, N), jnp.float32),
         jax.ShapeDtypeStruct((1, N), jnp.float32),
         jax.ShapeDtypeStruct((1, 16), jnp.float32)],
        nump0.reshape(1, 128, N), denp0.reshape(32, 2, N), wo, bo, aom)

    ex1, denp1 = sc_a1(sd, s1_1.reshape(-1), s2_1.reshape(-1),
                       c1.reshape(-1))
    nump1 = sc_b64(sd, ex1, zT1.reshape(-1))

    l1h = params["l1"]["heads"]
    w1, b1, av1 = head_wb(l1h[0])
    w2, b2, av2 = head_wb(l1h[1])
    a1m = jnp.stack([av1[:64], av1[64:]])
    a2m = jnp.stack([av2[:64], av2[64:]])
    zT2, s1_2, s2_2, c2 = _dense_call(
        mid2_body(64, 1, 2, 64),
        [jax.ShapeDtypeStruct((128, N), jnp.float32),
         jax.ShapeDtypeStruct((2, N), jnp.float32),
         jax.ShapeDtypeStruct((2, N), jnp.float32),
         jax.ShapeDtypeStruct((2, 16), jnp.float32)],
        nump1.reshape(2, 64, N), denp1.reshape(32, 1, N),
        w1, b1, a1m, w2, b2, a2m)

    ex2, denp2 = sc_a2(sd, s1_2.reshape(-1), s2_2.reshape(-1),
                       c2.reshape(-1))
    nump2 = sc_b128(sd, ex2, zT2.reshape(-1))

    l1o = params["l1"]["out"]
    wo, bo, avo = head_wb(l1o)
    aom = jnp.stack([avo[:64], avo[64:]])
    zT3, s1_3, s2_3, c3 = _dense_call(
        mid_body(128, 2, 1, 64),
        [jax.ShapeDtypeStruct((64, N), jnp.float32),
         jax.ShapeDtypeStruct((1, N), jnp.float32),
         jax.ShapeDtypeStruct((1, N), jnp.float32),
         jax.ShapeDtypeStruct((1, 16), jnp.float32)],
        nump2.reshape(1, 128, N), denp2.reshape(32, 2, N), wo, bo, aom)

    ex3, denp3 = sc_a1(sd, s1_3.reshape(-1), s2_3.reshape(-1),
                       c3.reshape(-1))
    nump3 = sc_b64(sd, ex3, zT3.reshape(-1))

    def d4_body(nump_ref, denp_ref, w1_ref, b1_ref, a1_ref,
                w2_ref, b2_ref, a2_ref, z_ref, s1_ref, s2_ref, c_ref):
        h_in = _merge(nump_ref[...], denp_ref[...], 64, 1, 2)
        for h, (w_ref, b_ref, a_ref) in enumerate(
                ((w1_ref, b1_ref, a1_ref), (w2_ref, b2_ref, a2_ref))):
            z = lax.dot_general(w_ref[...], h_in, (((0,), (0,)), ((), ())),
                                preferred_element_type=jnp.float32)
            z = z + b_ref[...][:, None]
            a = a_ref[...]
            s1, s2, c = _attn_scalars(z, a[:1], a[1:])
            z_ref[pl.ds(h * 8, 7), :] = z
            z_ref[pl.ds(h * 8 + 7, 1), :] = jnp.zeros((1, N), jnp.float32)
            s1_ref[pl.ds(h, 1), :] = s1
            s2_ref[pl.ds(h, 1), :] = s2
            c_ref[pl.ds(h, 1), :] = jnp.full((1, 16), c, jnp.float32)

    olh = params["outl"]["heads"]
    w1, b1, av1 = head_wb(olh[0])
    w2, b2, av2 = head_wb(olh[1])
    a1m = jnp.stack([av1[:7], av1[7:]])
    a2m = jnp.stack([av2[:7], av2[7:]])
    zT4, s1_4, s2_4, c4 = _dense_call(
        d4_body,
        [jax.ShapeDtypeStruct((16, N), jnp.float32),
         jax.ShapeDtypeStruct((2, N), jnp.float32),
         jax.ShapeDtypeStruct((2, N), jnp.float32),
         jax.ShapeDtypeStruct((2, 16), jnp.float32)],
        nump3.reshape(2, 64, N), denp3.reshape(32, 1, N),
        w1, b1, a1m, w2, b2, a2m)

    ex4, denp4 = sc_a2(sd, s1_4.reshape(-1), s2_4.reshape(-1),
                       c4.reshape(-1))
    nump4 = sc_b16(sd, ex4, zT4.reshape(-1))

    def d5_body(nump_ref, denp_ref, lw_ref, lb_ref, out_ref):
        num = jnp.sum(nump_ref[...], axis=0)
        den = jnp.sum(denp_ref[...], axis=0)
        den = jnp.maximum(den, 1e-16)
        o0 = num[0:7] / den[0:1]
        o1 = num[8:15] / den[1:2]
        hmean = 0.5 * (o0 + o1)
        logits = lax.dot_general(hmean, lw_ref[...], (((0,), (0,)), ((), ())),
                                 preferred_element_type=jnp.float32)
        logits = logits + lb_ref[...][None, :]
        m = jnp.max(logits, axis=1, keepdims=True)
        ez = jnp.exp(logits - m)
        out_ref[...] = ez / jnp.sum(ez, axis=1, keepdims=True)

    lin = params["outl"]["lin"]
    out = _dense_call(
        d5_body,
        jax.ShapeDtypeStruct((N, 7), jnp.float32),
        nump4.reshape(8, 16, N), denp4.reshape(32, 2, N),
        lin["W"], lin["b"])
    return out


# speedup vs baseline: 103.5811x; 1.0953x over previous
import functools

import jax
import jax.numpy as jnp
from jax import lax
from jax.experimental import pallas as pl
from jax.experimental.pallas import tpu as pltpu
from jax.experimental.pallas import tpu_sc as plsc

N = 10000
E = 320000
_SC_PARAMS = pltpu.CompilerParams(needs_layout_passes=False)
_MESH = plsc.VectorSubcoreMesh(core_axis_name="c", subcore_axis_name="s")


def _gelu_exp(x, cvec):
    xa = jnp.abs(x) * 0.7071067811865476
    t = 1.0 / (1.0 + 0.3275911 * xa)
    poly = t * (0.254829592 + t * (-0.284496736 + t * (1.421413741
                + t * (-1.453152027 + t * 1.061405429))))
    erf = 1.0 - poly * jnp.exp(-xa * xa)
    erf = jnp.where(x >= 0, erf, -erf)
    g = 0.5 * x * (1.0 + erf)
    return jnp.exp(g - cvec)


def _make_sc_a(H):
    ET = E // 32
    CH = 2000
    NCH = ET // CH

    @functools.partial(
        pl.kernel,
        out_type=[jax.ShapeDtypeStruct((H * E,), jnp.float32),
                  jax.ShapeDtypeStruct((32 * H, N), jnp.float32)],
        mesh=_MESH,
        compiler_params=_SC_PARAMS,
        scratch_types=[pltpu.VMEM((H, N), jnp.float32),
                       pltpu.VMEM((H, N), jnp.float32),
                       pltpu.VMEM((H, N), jnp.float32),
                       pltpu.VMEM((CH,), jnp.int32),
                       pltpu.VMEM((CH,), jnp.int32),
                       pltpu.VMEM((H * CH,), jnp.float32),
                       pltpu.VMEM((H * CH,), jnp.float32),
                       pltpu.VMEM((H * 16,), jnp.float32),
                       pltpu.SemaphoreType.DMA,
                       pltpu.SemaphoreType.DMA,
                       pltpu.SemaphoreType.DMA,
                       pltpu.SemaphoreType.DMA],
    )
    def sc_a(sd_hbm, s1_hbm, s2_hbm, c_hbm, zden_hbm, ex_hbm, den_hbm,
             s1t, s2t, den, sdb0, sdb1, exb0, exb1, cvb,
             semi0, semi1, semo0, semo1):
        cc = lax.axis_index("c")
        ss = lax.axis_index("s")
        wid = ss * 2 + cc
        sdbufs = (sdb0, sdb1)
        exbufs = (exb0, exb1)
        semis = (semi0, semi1)
        semos = (semo0, semo1)
        base0 = wid * ET

        def start_in(cidx, p):
            base = base0 + cidx * CH
            pltpu.async_copy(sd_hbm.at[pl.ds(base, CH)], sdbufs[p],
                             semis[p])

        def drain_in(cidx, p):
            base = base0 + cidx * CH
            pltpu.make_async_copy(sd_hbm.at[pl.ds(base, CH)], sdbufs[p],
                                  semis[p]).wait()

        def start_out(cidx, p):
            base = base0 + cidx * CH
            for h in range(H):
                pltpu.async_copy(exbufs[p].at[pl.ds(h * CH, CH)],
                                 ex_hbm.at[pl.ds(h * E + base, CH)],
                                 semos[p])

        def drain_out(cidx, p):
            base = base0 + cidx * CH
            for h in range(H):
                pltpu.make_async_copy(exbufs[p].at[pl.ds(h * CH, CH)],
                                      ex_hbm.at[pl.ds(h * E + base, CH)],
                                      semos[p]).wait()

        start_in(0, 0)
        start_in(1, 1)
        pltpu.sync_copy(s1_hbm, s1t)
        pltpu.sync_copy(s2_hbm, s2t)
        pltpu.sync_copy(c_hbm, cvb)
        pltpu.sync_copy(zden_hbm, den)

        for cidx in range(NCH):
            p = cidx % 2
            sdb, exb = sdbufs[p], exbufs[p]
            drain_in(cidx, p)
            if cidx + 2 < NCH:
                start_in(cidx + 2, p)
            if cidx >= 2:
                drain_out(cidx - 2, p)

            @functools.partial(plsc.parallel_loop, 0, CH // 16, unroll=4)
            def _(i):
                off = i * 16
                sd = sdb[pl.ds(off, 16)]
                sv = jnp.bitwise_and(sd, 0xFFFF)
                dv = lax.shift_right_logical(sd, 16)
                for h in range(H):
                    hvec = jnp.full((16,), h, jnp.int32)
                    a1 = plsc.load_gather(s1t, [hvec, sv])
                    a2 = plsc.load_gather(s2t, [hvec, dv])
                    ex = _gelu_exp(a1 + a2, cvb[pl.ds(h * 16, 16)])
                    exb[pl.ds(h * CH + off, 16)] = ex
                    plsc.addupdate_scatter(den, [hvec, dv], ex)

            start_out(cidx, p)
        drain_out(NCH - 2, (NCH - 2) % 2)
        drain_out(NCH - 1, (NCH - 1) % 2)
        pltpu.sync_copy(den, den_hbm.at[pl.ds(wid * H, H)])

    return sc_a


def _make_sc_b(F, H):
    K = F // 4
    T = 32 // K
    ET = E // T
    CH = 8000 if T < 8 else 4000
    NCH = ET // CH
    KH = K // H
    assert NCH % 2 == 0

    @functools.partial(
        pl.kernel,
        out_type=jax.ShapeDtypeStruct((T * F, N), jnp.float32),
        mesh=_MESH,
        compiler_params=_SC_PARAMS,
        scratch_types=[pltpu.VMEM((4, N), jnp.float32),
                       pltpu.VMEM((4, N), jnp.float32),
                       pltpu.VMEM((CH,), jnp.int32),
                       pltpu.VMEM((CH,), jnp.int32),
                       pltpu.VMEM((CH,), jnp.float32),
                       pltpu.VMEM((CH,), jnp.float32),
                       pltpu.SemaphoreType.DMA,
                       pltpu.SemaphoreType.DMA],
    )
    def sc_b(sd_hbm, ex_hbm, z_hbm, zacc_hbm, num_hbm,
             zc, acc, sdb0, sdb1, exb0, exb1, sem0, sem1):
        cc = lax.axis_index("c")
        ss = lax.axis_index("s")
        wid = ss * 2 + cc
        chunk = wid // T
        part = wid % T
        f0 = chunk * 4
        head = chunk // KH
        sdbufs = (sdb0, sdb1)
        exbufs = (exb0, exb1)
        sems = (sem0, sem1)
        ebase0 = part * ET

        def start(cidx, p):
            base = ebase0 + cidx * CH
            pltpu.async_copy(sd_hbm.at[pl.ds(base, CH)], sdbufs[p], sems[p])
            pltpu.async_copy(ex_hbm.at[pl.ds(head * E + base, CH)],
                             exbufs[p], sems[p])

        def drain(cidx, p):
            base = ebase0 + cidx * CH
            pltpu.make_async_copy(sd_hbm.at[pl.ds(base, CH)],
                                  sdbufs[p], sems[p]).wait()
            pltpu.make_async_copy(ex_hbm.at[pl.ds(head * E + base, CH)],
                                  exbufs[p], sems[p]).wait()

        start(0, 0)
        start(1, 1)
        pltpu.sync_copy(z_hbm.at[pl.ds(f0, 4)], zc)
        pltpu.sync_copy(zacc_hbm, acc)

        def compute(p):
            sdb, exb = sdbufs[p], exbufs[p]

            @functools.partial(plsc.parallel_loop, 0, CH // 16, unroll=8)
            def _(i):
                off = i * 16
                sd = sdb[pl.ds(off, 16)]
                sv = jnp.bitwise_and(sd, 0xFFFF)
                dv = lax.shift_right_logical(sd, 16)
                w = exb[pl.ds(off, 16)]
                for r in range(4):
                    rvec = jnp.full((16,), r, jnp.int32)
                    gth = plsc.load_gather(zc, [rvec, sv])
                    plsc.addupdate_scatter(acc, [rvec, dv], gth * w)

        def chunk_body(g, _):
            for p in range(2):
                cidx = g * 2 + p
                drain(cidx, p)
                compute(p)
                start(cidx + 2, p)
            return 0
        lax.fori_loop(0, NCH // 2 - 1, chunk_body, 0)
        for p in range(2):
            drain(NCH - 2 + p, p)
            compute(p)
        pltpu.sync_copy(acc, num_hbm.at[pl.ds(part * F + f0, 4)])

    return sc_b


def _attn_scalars(z, a1, a2):
    s1 = lax.dot_general(a1, z, (((1,), (0,)), ((), ())),
                         preferred_element_type=jnp.float32)
    s2 = lax.dot_general(a2, z, (((1,), (0,)), ((), ())),
                         preferred_element_type=jnp.float32)
    c = jnp.clip(jnp.max(s1) + jnp.max(s2), 0.0, 30.0)
    return s1, s2, c


def _merge(nump, denp, F, H, T):
    num = nump[0:F]
    for t in range(1, T):
        num = num + nump[t * F:(t + 1) * F]
    den = denp[0:H]
    for w in range(1, 32):
        den = den + denp[w * H:(w + 1) * H]
    den = jnp.maximum(den, 1e-16)
    FH = F // H
    dens = [jnp.broadcast_to(den[h:h + 1], (FH, N)) for h in range(H)]
    return num / jnp.concatenate(dens, axis=0)


def _dense_call(body, out_shapes, *inputs):
    return pl.pallas_call(
        body,
        out_shape=out_shapes,
    )(*inputs)


def kernel(feature, params, edge_index):

    def head_wb(p):
        return p["W"], p["b"], p["a"][:, 0]

    z1 = jnp.zeros((1, N), jnp.float32)
    z2 = jnp.zeros((2, N), jnp.float32)
    z4 = jnp.zeros((4, N), jnp.float32)

    def d0_body(feat_ref, ei_ref, w1_ref, b1_ref, a1_ref,
                w2_ref, b2_ref, a2_ref,
                z_ref, s1_ref, s2_ref, c_ref, sd_ref):
        ei = ei_ref[...]
        sd_ref[...] = jnp.bitwise_or(ei[0:1],
                                     lax.shift_left(ei[1:2], 16))
        feat = feat_ref[...]
        for h, (w_ref, b_ref, a_ref) in enumerate(
                ((w1_ref, b1_ref, a1_ref), (w2_ref, b2_ref, a2_ref))):
            z = lax.dot_general(w_ref[...], feat, (((0,), (1,)), ((), ())),
                                preferred_element_type=jnp.float32)
            z = z + b_ref[...][:, None]
            a = a_ref[...]
            s1, s2, c = _attn_scalars(z, a[:1], a[1:])
            z_ref[pl.ds(h * 64, 64), :] = z
            s1_ref[pl.ds(h, 1), :] = s1
            s2_ref[pl.ds(h, 1), :] = s2
            c_ref[pl.ds(h * 16, 16)] = jnp.full((16,), c, jnp.float32)

    l0h = params["l0"]["heads"]
    w1, b1, av1 = head_wb(l0h[0])
    w2, b2, av2 = head_wb(l0h[1])
    a1m = jnp.stack([av1[:64], av1[64:]])
    a2m = jnp.stack([av2[:64], av2[64:]])
    zT0, s1_0, s2_0, c0, sd2 = _dense_call(
        d0_body,
        [jax.ShapeDtypeStruct((128, N), jnp.float32),
         jax.ShapeDtypeStruct((2, N), jnp.float32),
         jax.ShapeDtypeStruct((2, N), jnp.float32),
         jax.ShapeDtypeStruct((32,), jnp.float32),
         jax.ShapeDtypeStruct((1, E), jnp.int32)],
        feature, edge_index, w1, b1, a1m, w2, b2, a2m)
    sd = sd2.reshape(-1)

    sc_a2 = _make_sc_a(2)
    sc_a1 = _make_sc_a(1)
    sc_b128 = _make_sc_b(128, 2)
    sc_b64 = _make_sc_b(64, 1)
    sc_b16 = _make_sc_b(16, 2)

    ex0, denp0 = sc_a2(sd, s1_0, s2_0, c0, z2)
    nump0 = sc_b128(sd, ex0, zT0, z4)

    def mid_body(F_in, H_in, T_in, FH_out):
        def body(nump_ref, denp_ref, w_ref, b_ref, a_ref,
                 z_ref, s1_ref, s2_ref, c_ref):
            h_in = _merge(nump_ref[...], denp_ref[...], F_in, H_in, T_in)
            z = lax.dot_general(w_ref[...], h_in, (((0,), (0,)), ((), ())),
                                preferred_element_type=jnp.float32)
            z = z + b_ref[...][:, None]
            a = a_ref[...]
            s1, s2, c = _attn_scalars(z, a[:1], a[1:])
            z_ref[...] = z
            s1_ref[...] = s1
            s2_ref[...] = s2
            c_ref[...] = jnp.full((16,), c, jnp.float32)
        return body

    def mid2_body(F_in, H_in, T_in, FH_out):
        def body(nump_ref, denp_ref, w1_ref, b1_ref, a1_ref,
                 w2_ref, b2_ref, a2_ref, z_ref, s1_ref, s2_ref, c_ref):
            h_in = _merge(nump_ref[...], denp_ref[...], F_in, H_in, T_in)
            for h, (w_ref, b_ref, a_ref) in enumerate(
                    ((w1_ref, b1_ref, a1_ref), (w2_ref, b2_ref, a2_ref))):
                z = lax.dot_general(w_ref[...], h_in,
                                    (((0,), (0,)), ((), ())),
                                    preferred_element_type=jnp.float32)
                z = z + b_ref[...][:, None]
                a = a_ref[...]
                s1, s2, c = _attn_scalars(z, a[:1], a[1:])
                z_ref[pl.ds(h * FH_out, FH_out), :] = z
                s1_ref[pl.ds(h, 1), :] = s1
                s2_ref[pl.ds(h, 1), :] = s2
                c_ref[pl.ds(h * 16, 16)] = jnp.full((16,), c, jnp.float32)
        return body

    l0o = params["l0"]["out"]
    wo, bo, avo = head_wb(l0o)
    aom = jnp.stack([avo[:64], avo[64:]])
    zT1, s1_1, s2_1, c1 = _dense_call(
        mid_body(128, 2, 1, 64),
        [jax.ShapeDtypeStruct((64, N), jnp.float32),
         jax.ShapeDtypeStruct((1, N), jnp.float32),
         jax.ShapeDtypeStruct((1, N), jnp.float32),
         jax.ShapeDtypeStruct((16,), jnp.float32)],
        nump0, denp0, wo, bo, aom)

    ex1, denp1 = sc_a1(sd, s1_1, s2_1, c1, z1)
    nump1 = sc_b64(sd, ex1, zT1, z4)

    l1h = params["l1"]["heads"]
    w1, b1, av1 = head_wb(l1h[0])
    w2, b2, av2 = head_wb(l1h[1])
    a1m = jnp.stack([av1[:64], av1[64:]])
    a2m = jnp.stack([av2[:64], av2[64:]])
    zT2, s1_2, s2_2, c2 = _dense_call(
        mid2_body(64, 1, 2, 64),
        [jax.ShapeDtypeStruct((128, N), jnp.float32),
         jax.ShapeDtypeStruct((2, N), jnp.float32),
         jax.ShapeDtypeStruct((2, N), jnp.float32),
         jax.ShapeDtypeStruct((32,), jnp.float32)],
        nump1, denp1, w1, b1, a1m, w2, b2, a2m)

    ex2, denp2 = sc_a2(sd, s1_2, s2_2, c2, z2)
    nump2 = sc_b128(sd, ex2, zT2, z4)

    l1o = params["l1"]["out"]
    wo, bo, avo = head_wb(l1o)
    aom = jnp.stack([avo[:64], avo[64:]])
    zT3, s1_3, s2_3, c3 = _dense_call(
        mid_body(128, 2, 1, 64),
        [jax.ShapeDtypeStruct((64, N), jnp.float32),
         jax.ShapeDtypeStruct((1, N), jnp.float32),
         jax.ShapeDtypeStruct((1, N), jnp.float32),
         jax.ShapeDtypeStruct((16,), jnp.float32)],
        nump2, denp2, wo, bo, aom)

    ex3, denp3 = sc_a1(sd, s1_3, s2_3, c3, z1)
    nump3 = sc_b64(sd, ex3, zT3, z4)

    def d4_body(nump_ref, denp_ref, w1_ref, b1_ref, a1_ref,
                w2_ref, b2_ref, a2_ref, z_ref, s1_ref, s2_ref, c_ref):
        h_in = _merge(nump_ref[...], denp_ref[...], 64, 1, 2)
        for h, (w_ref, b_ref, a_ref) in enumerate(
                ((w1_ref, b1_ref, a1_ref), (w2_ref, b2_ref, a2_ref))):
            z = lax.dot_general(w_ref[...], h_in, (((0,), (0,)), ((), ())),
                                preferred_element_type=jnp.float32)
            z = z + b_ref[...][:, None]
            a = a_ref[...]
            s1, s2, c = _attn_scalars(z, a[:1], a[1:])
            z_ref[pl.ds(h * 8, 7), :] = z
            z_ref[pl.ds(h * 8 + 7, 1), :] = jnp.zeros((1, N), jnp.float32)
            s1_ref[pl.ds(h, 1), :] = s1
            s2_ref[pl.ds(h, 1), :] = s2
            c_ref[pl.ds(h * 16, 16)] = jnp.full((16,), c, jnp.float32)

    olh = params["outl"]["heads"]
    w1, b1, av1 = head_wb(olh[0])
    w2, b2, av2 = head_wb(olh[1])
    a1m = jnp.stack([av1[:7], av1[7:]])
    a2m = jnp.stack([av2[:7], av2[7:]])
    zT4, s1_4, s2_4, c4 = _dense_call(
        d4_body,
        [jax.ShapeDtypeStruct((16, N), jnp.float32),
         jax.ShapeDtypeStruct((2, N), jnp.float32),
         jax.ShapeDtypeStruct((2, N), jnp.float32),
         jax.ShapeDtypeStruct((32,), jnp.float32)],
        nump3, denp3, w1, b1, a1m, w2, b2, a2m)

    ex4, denp4 = sc_a2(sd, s1_4, s2_4, c4, z2)
    nump4 = sc_b16(sd, ex4, zT4, z4)

    def d5_body(nump_ref, denp_ref, lw_ref, lb_ref, out_ref):
        nump = nump_ref[...]
        denp = denp_ref[...]
        num = nump[0:16]
        for t in range(1, 8):
            num = num + nump[t * 16:(t + 1) * 16]
        den = denp[0:2]
        for w in range(1, 32):
            den = den + denp[w * 2:(w + 1) * 2]
        den = jnp.maximum(den, 1e-16)
        o0 = num[0:7] / den[0:1]
        o1 = num[8:15] / den[1:2]
        hmean = 0.5 * (o0 + o1)
        logits = lax.dot_general(hmean, lw_ref[...], (((0,), (0,)), ((), ())),
                                 preferred_element_type=jnp.float32)
        logits = logits + lb_ref[...][None, :]
        m = jnp.max(logits, axis=1, keepdims=True)
        ez = jnp.exp(logits - m)
        out_ref[...] = ez / jnp.sum(ez, axis=1, keepdims=True)

    lin = params["outl"]["lin"]
    out = _dense_call(
        d5_body,
        jax.ShapeDtypeStruct((N, 7), jnp.float32),
        nump4, denp4, lin["W"], lin["b"])
    return out
